# Initial kernel scaffold; baseline (speedup 1.0000x reference)
import jax
import jax.numpy as jnp
from jax.experimental import pallas as pl


def kernel(aff_x, aff_adj, drug_x, drug_edge_index, drug_batch, target_x, target_edge_index, target_batch, params, num_drug, num_target):
    raise NotImplementedError("write your pallas kernel here")


# speedup vs baseline: 1.0336x; 1.0336x over previous
import functools

import jax
import jax.numpy as jnp
from jax import lax
from jax.experimental import pallas as pl
from jax.experimental.pallas import tpu as pltpu

F32 = jnp.float32


def _rowsum_body(a_ref, o_ref):
    s = jnp.sum(a_ref[...], axis=1, keepdims=True)
    o_ref[...] = jnp.broadcast_to(s, o_ref.shape)


def _aff_rowsum(a_pad):
    m = a_pad.shape[0]
    bm = 256
    return pl.pallas_call(
        _rowsum_body,
        grid=(m // bm,),
        in_specs=[pl.BlockSpec((bm, m), lambda i: (i, 0))],
        out_specs=pl.BlockSpec((bm, 8), lambda i: (i, 0)),
        out_shape=jax.ShapeDtypeStruct((m, 8), F32),
    )(a_pad)


def _mm_scale_body(x_ref, w_ref, s_ref, o_ref):
    d = lax.rsqrt(jnp.clip(s_ref[...][:, :1], 1.0, None))
    t = jnp.dot(x_ref[...], w_ref[...], preferred_element_type=F32)
    o_ref[...] = t * d


def _mm_scale(x, w, s, bm=256):
    m, k = x.shape
    n = w.shape[1]
    return pl.pallas_call(
        _mm_scale_body,
        grid=(m // bm,),
        in_specs=[
            pl.BlockSpec((bm, k), lambda i: (i, 0)),
            pl.BlockSpec((k, n), lambda i: (0, 0)),
            pl.BlockSpec((bm, 8), lambda i: (i, 0)),
        ],
        out_specs=pl.BlockSpec((bm, n), lambda i: (i, 0)),
        out_shape=jax.ShapeDtypeStruct((m, n), F32),
    )(x, w, s)


def _amm_body(a_ref, y_ref, s_ref, b_ref, o_ref):
    t = jnp.dot(a_ref[...], y_ref[...], preferred_element_type=F32)
    d = lax.rsqrt(jnp.clip(s_ref[...][:, :1], 1.0, None))
    o_ref[...] = jax.nn.relu(t * d + b_ref[...])


def _amm(a, y, s, b, bm=256):
    m = a.shape[0]
    k = y.shape[0]
    n = y.shape[1]
    return pl.pallas_call(
        _amm_body,
        grid=(m // bm,),
        in_specs=[
            pl.BlockSpec((bm, k), lambda i: (i, 0)),
            pl.BlockSpec((k, n), lambda i: (0, 0)),
            pl.BlockSpec((bm, 8), lambda i: (i, 0)),
            pl.BlockSpec((1, n), lambda i: (0, 0)),
        ],
        out_specs=pl.BlockSpec((bm, n), lambda i: (i, 0)),
        out_shape=jax.ShapeDtypeStruct((m, n), F32),
    )(a, y, s, b)


def _dense_gcn_tc(aff_x, aff_adj, p):
    n = aff_adj.shape[0]
    m = ((n + 255) // 256) * 256
    a_pad = jnp.pad(aff_adj, ((0, m - n), (0, m - n)))
    x_pad = jnp.pad(aff_x, ((0, m - n), (0, 0)))
    s = _aff_rowsum(a_pad)
    y = _mm_scale(x_pad, p['aff_W0'], s)
    h1 = _amm(a_pad, y, s, p['aff_b0'].reshape(1, -1))
    y2 = _mm_scale(h1, p['aff_W1'], s)
    h2 = _amm(a_pad, y2, s, p['aff_b1'].reshape(1, -1))
    return h2[:n]


def _elu(x):
    return jnp.where(x > 0, x, jnp.exp(x) - 1.0)


def _contrast_body(za_ref, zb_ref, p1_ref, pb1_ref, p2_ref, pb2_ref,
                   wq_ref, bq_ref, wk_ref, bk_ref, wv_ref, bv_ref, o_ref):
    za = za_ref[...]
    zb = zb_ref[...]
    p1 = p1_ref[...]
    p2 = p2_ref[...]
    za_p = jnp.dot(_elu(jnp.dot(za, p1, preferred_element_type=F32) + pb1_ref[...]),
                   p2, preferred_element_type=F32) + pb2_ref[...]
    zb_p = jnp.dot(_elu(jnp.dot(zb, p1, preferred_element_type=F32) + pb1_ref[...]),
                   p2, preferred_element_type=F32) + pb2_ref[...]
    scale = jnp.sqrt(za_p.shape[1] / 2.0).astype(F32)

    def ca(q_in, k_in, v_in):
        q = jnp.dot(q_in, wq_ref[...], preferred_element_type=F32) + bq_ref[...]
        k = jnp.dot(k_in, wk_ref[...], preferred_element_type=F32) + bk_ref[...]
        v = jnp.dot(v_in, wv_ref[...], preferred_element_type=F32) + bv_ref[...]
        logits = lax.dot_general(q, k, (((1,), (1,)), ((), ())),
                                 preferred_element_type=F32) / scale
        logits = logits - jnp.max(logits, axis=-1, keepdims=True)
        e = jnp.exp(logits)
        a = e / jnp.sum(e, axis=-1, keepdims=True)
        return jnp.dot(a, v, preferred_element_type=F32)

    o_ref[...] = jnp.concatenate([ca(za_p, zb_p, zb_p), ca(zb_p, za_p, za_p)], axis=1)


def _contrast_tc(za, zb, p, pre):
    m = za.shape[0]
    args = (za, zb, p[pre + 'P1'], p[pre + 'pb1'].reshape(1, -1),
            p[pre + 'P2'], p[pre + 'pb2'].reshape(1, -1),
            p[pre + 'Wq'], p[pre + 'bq'].reshape(1, -1),
            p[pre + 'Wk'], p[pre + 'bk'].reshape(1, -1),
            p[pre + 'Wv'], p[pre + 'bv'].reshape(1, -1))
    return pl.pallas_call(
        _contrast_body,
        out_shape=jax.ShapeDtypeStruct((m, 256), F32),
    )(*args)


def _gcn_graph(x, edge_index, batch, w0, b0, w1, b1, g):
    n = x.shape[0]
    loop = jnp.arange(n, dtype=edge_index.dtype)
    row = jnp.concatenate([edge_index[0], loop])
    col = jnp.concatenate([edge_index[1], loop])
    w = jnp.ones(row.shape[0], dtype=F32)
    deg = jax.ops.segment_sum(w, col, num_segments=n)
    dis = jnp.where(deg > 0, lax.rsqrt(deg), 0.0)
    nrm = dis[row] * dis[col]

    h = x @ w0
    h = jax.nn.relu(jax.ops.segment_sum(h[row] * nrm[:, None], col, num_segments=n) + b0)
    h = h @ w1
    h = jax.nn.relu(jax.ops.segment_sum(h[row] * nrm[:, None], col, num_segments=n) + b1)
    s = jax.ops.segment_sum(h, batch, num_segments=g)
    c = jax.ops.segment_sum(jnp.ones((n,), F32), batch, num_segments=g)
    return s / jnp.clip(c, 1.0, None)[:, None]


def kernel(aff_x, aff_adj, drug_x, drug_edge_index, drug_batch,
           target_x, target_edge_index, target_batch, params, num_drug, num_target):
    nd, nt = 1000, 1500
    p = params
    aff_emb = _dense_gcn_tc(aff_x, aff_adj, p)
    drug_emb = _gcn_graph(drug_x, drug_edge_index, drug_batch,
                          p['d_W0'], p['d_b0'], p['d_W1'], p['d_b1'], nd)
    target_emb = _gcn_graph(target_x, target_edge_index, target_batch,
                            p['t_W0'], p['t_b0'], p['t_W1'], p['t_b1'], nt)
    drug_out = _contrast_tc(aff_emb[:nd], drug_emb, p, 'dc_')
    target_out = _contrast_tc(aff_emb[nd:], target_emb, p, 'tc_')
    drug_out = drug_out + jnp.asarray(num_drug - nd, dtype=drug_out.dtype)
    target_out = target_out + jnp.asarray(num_target - nt, dtype=target_out.dtype)
    return (drug_out, target_out)


# speedup vs baseline: 2.8480x; 2.7553x over previous
import functools

import jax
import jax.numpy as jnp
from jax import lax
from jax.experimental import pallas as pl
from jax.experimental.pallas import tpu as pltpu
from jax.experimental.pallas import tpu_sc as plsc

F32 = jnp.float32
I32 = jnp.int32
_NC = 2
_NS = 16
_KG = 4


def _sc_mesh():
    return plsc.VectorSubcoreMesh(core_axis_name="c", subcore_axis_name="s",
                                  num_cores=_NC, num_subcores=_NS)


def _rpt8(n):
    return -(-(-(-(n + 16) // 16)) // 8) * 8


def _deg_sc(col3_d, col3_t, ones128, zeros512, nd, nt):
    rpt_d = _rpt8(nd)
    rpt_t = _rpt8(nt)
    n_alloc = 16 * max(rpt_d, rpt_t)
    gd = col3_d.shape[0] // _NS
    gt = col3_t.shape[0] // _NS

    @functools.partial(
        pl.kernel, mesh=_sc_mesh(),
        compiler_params=pltpu.CompilerParams(use_tc_tiling_on_sc=False),
        out_type=[jax.ShapeDtypeStruct((nd, 16), F32),
                  jax.ShapeDtypeStruct((nt, 16), F32)],
        scratch_types=[
            pltpu.VMEM((_KG, 128), I32),
            pltpu.VMEM((128, 16), F32),
            pltpu.VMEM_SHARED((n_alloc, 16), F32),
        ],
    )
    def k(cd_hbm, ct_hbm, ones_hbm, z_hbm, dd_hbm, dt_hbm, idx_v, ones_v, acc):
        c = lax.axis_index("c")
        s = lax.axis_index("s")
        pltpu.sync_copy(ones_hbm, ones_v)
        for n, rpt, col_hbm, ngrp, out_hbm, core in (
                (nd, rpt_d, cd_hbm, gd, dd_hbm, 0),
                (nt, rpt_t, ct_hbm, gt, dt_hbm, 1)):
            @pl.when(c == core)
            def _():
                row0 = s * rpt
                zfull, ztail = rpt // 512, rpt % 512
                for zi in range(zfull):
                    pltpu.sync_copy(z_hbm,
                                    acc.at[pl.ds(row0 + zi * 512, 512)])
                if ztail:
                    pltpu.sync_copy(z_hbm.at[pl.ds(0, ztail)],
                                    acc.at[pl.ds(row0 + zfull * 512, ztail)])
                plsc.subcore_barrier()

                def grp(g, carry):
                    pltpu.sync_copy(col_hbm.at[s * ngrp + g], idx_v)
                    for j in range(_KG):
                        pltpu.sync_copy(ones_v, acc.at[idx_v.at[j]], add=True)
                    return carry
                lax.fori_loop(0, ngrp, grp, 0)
                plsc.subcore_barrier()

                last = n - 15 * rpt
                @pl.when(s < _NS - 1)
                def _():
                    pltpu.sync_copy(acc.at[pl.ds(row0, rpt)],
                                    out_hbm.at[pl.ds(row0, rpt)])
                @pl.when(s == _NS - 1)
                def _():
                    pltpu.sync_copy(acc.at[pl.ds(row0, last)],
                                    out_hbm.at[pl.ds(row0, last)])
                plsc.subcore_barrier()

    return k(col3_d, col3_t, ones128, zeros512)


def _spmm_sc(p2d, row3, col3, zeros512, n, f, nchunk):
    rpt = _rpt8(n)
    n_alloc = 16 * rpt
    ngrp = row3.shape[0] // _NS
    npass = nchunk // _NC

    @functools.partial(
        pl.kernel, mesh=_sc_mesh(),
        compiler_params=pltpu.CompilerParams(use_tc_tiling_on_sc=False),
        out_type=jax.ShapeDtypeStruct((nchunk * n, f), F32),
        scratch_types=[
            pltpu.VMEM((_KG, 128), I32),
            pltpu.VMEM((_KG, 128), I32),
            pltpu.VMEM((_KG, 128, f), F32),
            pltpu.VMEM_SHARED((n_alloc, f), F32),
            pltpu.SemaphoreType.DMA,
        ],
    )
    def k(p_hbm, row_hbm, col_hbm, z_hbm, s_hbm, idxr, idxc, buf, acc, sem):
        c = lax.axis_index("c")
        s = lax.axis_index("s")
        row0 = s * rpt
        zfull, ztail = rpt // 512, rpt % 512
        last = n - 15 * rpt

        def one_pass(ppass, carry):
            chunk = ppass * _NC + c
            off = chunk * n
            for zi in range(zfull):
                pltpu.sync_copy(z_hbm, acc.at[pl.ds(row0 + zi * 512, 512)])
            if ztail:
                pltpu.sync_copy(z_hbm.at[pl.ds(0, ztail)],
                                acc.at[pl.ds(row0 + zfull * 512, ztail)])
            plsc.subcore_barrier()

            def grp(g, carry2):
                base = s * ngrp + g
                pltpu.sync_copy(row_hbm.at[base], idxr)
                pltpu.sync_copy(col_hbm.at[base], idxc)
                offv = jnp.full((16,), off, I32)
                for j in range(_KG):
                    for jj in range(8):
                        sl = pl.ds(jj * 16, 16)
                        idxr[j, sl] = idxr[j, sl] + offv
                handles = [pltpu.async_copy(p_hbm.at[idxr.at[j]], buf.at[j], sem)
                           for j in range(_KG)]
                for h in handles:
                    h.wait()
                for j in range(_KG):
                    pltpu.sync_copy(buf.at[j], acc.at[idxc.at[j]], add=True)
                return carry2
            lax.fori_loop(0, ngrp, grp, 0)
            plsc.subcore_barrier()

            @pl.when(s < _NS - 1)
            def _():
                pltpu.sync_copy(acc.at[pl.ds(row0, rpt)],
                                s_hbm.at[pl.ds(off + row0, rpt)])
            @pl.when(s == _NS - 1)
            def _():
                pltpu.sync_copy(acc.at[pl.ds(row0, last)],
                                s_hbm.at[pl.ds(off + row0, last)])
            plsc.subcore_barrier()
            return carry

        lax.fori_loop(0, npass, one_pass, 0)

    return k(p2d, row3, col3, zeros512)


def _gmp_sc(h, batch3, zeros_g, zeros_c, ones128, n, g, rpw, w_partial, psize):
    g_pt = _rpt8(g)
    g_alloc = 16 * g_pt
    g_up = -(-g // 8) * 8
    nf_max = rpw // 128

    @functools.partial(
        pl.kernel, mesh=_sc_mesh(),
        compiler_params=pltpu.CompilerParams(use_tc_tiling_on_sc=False),
        out_type=[jax.ShapeDtypeStruct((_NC * g_up, 256), F32),
                  jax.ShapeDtypeStruct((_NC * g_up, 16), F32)],
        scratch_types=[
            pltpu.VMEM((128, 256), F32),
            pltpu.VMEM((128, 16), F32),
            pltpu.VMEM((nf_max, 128), I32),
            pltpu.VMEM_SHARED((g_alloc, 256), F32),
            pltpu.VMEM_SHARED((g_alloc, 16), F32),
        ],
    )
    def k(h_hbm, b_hbm, zg_hbm, zc_hbm, ones_hbm, s_hbm, c_hbm,
          buf, ones_v, idxb, accs, accc):
        c = lax.axis_index("c")
        s = lax.axis_index("s")
        w = c * _NS + s
        pltpu.sync_copy(ones_hbm, ones_v)
        grow0 = s * g_pt
        pltpu.sync_copy(zg_hbm.at[pl.ds(0, g_pt)], accs.at[pl.ds(grow0, g_pt)])
        pltpu.sync_copy(zc_hbm.at[pl.ds(0, g_pt)], accc.at[pl.ds(grow0, g_pt)])
        plsc.subcore_barrier()

        base = w * rpw
        nfull = jnp.clip((n - base) // 128, 0, nf_max)
        pltpu.sync_copy(b_hbm.at[w], idxb)

        def chunk(j, carry):
            r0 = base + j * 128
            pltpu.sync_copy(h_hbm.at[pl.ds(r0, 128)], buf)
            pltpu.sync_copy(buf, accs.at[idxb.at[j]], add=True)
            pltpu.sync_copy(ones_v, accc.at[idxb.at[j]], add=True)
            return carry
        lax.fori_loop(0, nfull, chunk, 0)

        if psize:
            p_j = (n - w_partial * rpw) // 128

            @pl.when(w == w_partial)
            def _():
                r0 = w_partial * rpw + p_j * 128
                pltpu.sync_copy(h_hbm.at[pl.ds(r0, psize)],
                                buf.at[pl.ds(0, psize)])
                pltpu.sync_copy(buf, accs.at[idxb.at[p_j]], add=True)
                pltpu.sync_copy(ones_v, accc.at[idxb.at[p_j]], add=True)
        plsc.subcore_barrier()

        glast = g - 15 * g_pt
        @pl.when(s < _NS - 1)
        def _():
            pltpu.sync_copy(accs.at[pl.ds(grow0, g_pt)],
                            s_hbm.at[pl.ds(c * g_up + grow0, g_pt)])
            pltpu.sync_copy(accc.at[pl.ds(grow0, g_pt)],
                            c_hbm.at[pl.ds(c * g_up + grow0, g_pt)])
        @pl.when(s == _NS - 1)
        def _():
            pltpu.sync_copy(accs.at[pl.ds(grow0, glast)],
                            s_hbm.at[pl.ds(c * g_up + grow0, glast)])
            pltpu.sync_copy(accc.at[pl.ds(grow0, glast)],
                            c_hbm.at[pl.ds(c * g_up + grow0, glast)])

    return k(h, batch3, zeros_g, zeros_c, ones128)


def _rowsum_body(a_ref, o_ref):
    o_ref[...] = jnp.broadcast_to(jnp.sum(a_ref[...], axis=1, keepdims=True),
                                  o_ref.shape)


def _aff_rowsum(a_pad):
    m = a_pad.shape[0]
    bm = 256
    return pl.pallas_call(
        _rowsum_body,
        grid=(m // bm,),
        in_specs=[pl.BlockSpec((bm, m), lambda i: (i, 0))],
        out_specs=pl.BlockSpec((bm, 8), lambda i: (i, 0)),
        out_shape=jax.ShapeDtypeStruct((m, 8), F32),
    )(a_pad)


def _mm_scale_body(x_ref, w_ref, s_ref, o_ref):
    d = lax.rsqrt(jnp.clip(s_ref[...][:, :1], 1.0, None))
    o_ref[...] = jnp.dot(x_ref[...], w_ref[...], preferred_element_type=F32) * d


def _mm_scale(x, w, s, bm=256):
    m, kk = x.shape
    n = w.shape[1]
    return pl.pallas_call(
        _mm_scale_body,
        grid=(m // bm,),
        in_specs=[
            pl.BlockSpec((bm, kk), lambda i: (i, 0)),
            pl.BlockSpec((kk, n), lambda i: (0, 0)),
            pl.BlockSpec((bm, 8), lambda i: (i, 0)),
        ],
        out_specs=pl.BlockSpec((bm, n), lambda i: (i, 0)),
        out_shape=jax.ShapeDtypeStruct((m, n), F32),
    )(x, w, s)


def _amm_body(a_ref, y_ref, s_ref, b_ref, o_ref):
    t = jnp.dot(a_ref[...], y_ref[...], preferred_element_type=F32)
    d = lax.rsqrt(jnp.clip(s_ref[...][:, :1], 1.0, None))
    o_ref[...] = jax.nn.relu(t * d + b_ref[...])


def _amm(a, y, s, b, bm=256):
    m = a.shape[0]
    kk, n = y.shape
    return pl.pallas_call(
        _amm_body,
        grid=(m // bm,),
        in_specs=[
            pl.BlockSpec((bm, kk), lambda i: (i, 0)),
            pl.BlockSpec((kk, n), lambda i: (0, 0)),
            pl.BlockSpec((bm, 8), lambda i: (i, 0)),
            pl.BlockSpec((1, n), lambda i: (0, 0)),
        ],
        out_specs=pl.BlockSpec((bm, n), lambda i: (i, 0)),
        out_shape=jax.ShapeDtypeStruct((m, n), F32),
    )(a, y, s, b)


def _dense_gcn_tc(aff_x, aff_adj, p):
    n = aff_adj.shape[0]
    m = ((n + 255) // 256) * 256
    a_pad = jnp.pad(aff_adj, ((0, m - n), (0, m - n)))
    x_pad = jnp.pad(aff_x, ((0, m - n), (0, 0)))
    s = _aff_rowsum(a_pad)
    y = _mm_scale(x_pad, p['aff_W0'], s)
    h1 = _amm(a_pad, y, s, p['aff_b0'].reshape(1, -1))
    y2 = _mm_scale(h1, p['aff_W1'], s)
    h2 = _amm(a_pad, y2, s, p['aff_b1'].reshape(1, -1))
    return h2[:n]


def _node_mm_body(x_ref, w_ref, cnt_ref, o_ref, *, nchunk, f):
    d = lax.rsqrt(cnt_ref[...][:, :1] + 1.0)
    t = jnp.dot(x_ref[...], w_ref[...], preferred_element_type=F32) * d
    for kk in range(nchunk):
        o_ref[kk] = t[:, kk * f:(kk + 1) * f]


def _node_mm(x, w, cnt, nchunk, f, bm=1000):
    n, kdim = x.shape
    body = functools.partial(_node_mm_body, nchunk=nchunk, f=f)
    out = pl.pallas_call(
        body,
        grid=(n // bm,),
        in_specs=[
            pl.BlockSpec((bm, kdim), lambda i: (i, 0)),
            pl.BlockSpec((kdim, w.shape[1]), lambda i: (0, 0)),
            pl.BlockSpec((bm, 16), lambda i: (i, 0)),
        ],
        out_specs=pl.BlockSpec((nchunk, bm, f), lambda i: (0, i, 0)),
        out_shape=jax.ShapeDtypeStruct((nchunk, n, f), F32),
    )(x, w, cnt)
    return out.reshape(nchunk * n, f)


def _layer2_body(s3_ref, p3_ref, cnt_ref, w_ref, b_ref, o_ref, *, nchunk, f):
    d = lax.rsqrt(cnt_ref[...][:, :1] + 1.0)
    h = jnp.concatenate([s3_ref[kk] + p3_ref[kk] for kk in range(nchunk)], axis=1)
    h = jax.nn.relu(h * d + b_ref[...])
    t = jnp.dot(h, w_ref[...], preferred_element_type=F32) * d
    for kk in range(nchunk):
        o_ref[kk] = t[:, kk * f:(kk + 1) * f]


def _layer2_mm(s2d, p2d, cnt, w, b, nchunk, f, bm=1000):
    n = cnt.shape[0]
    s3 = s2d.reshape(nchunk, n, f)
    p3 = p2d.reshape(nchunk, n, f)
    body = functools.partial(_layer2_body, nchunk=nchunk, f=f)
    out = pl.pallas_call(
        body,
        grid=(n // bm,),
        in_specs=[
            pl.BlockSpec((nchunk, bm, f), lambda i: (0, i, 0)),
            pl.BlockSpec((nchunk, bm, f), lambda i: (0, i, 0)),
            pl.BlockSpec((bm, 16), lambda i: (i, 0)),
            pl.BlockSpec((w.shape[0], w.shape[1]), lambda i: (0, 0)),
            pl.BlockSpec((1, w.shape[1]), lambda i: (0, 0)),
        ],
        out_specs=pl.BlockSpec((nchunk, bm, f), lambda i: (0, i, 0)),
        out_shape=jax.ShapeDtypeStruct((nchunk, n, f), F32),
    )(s3, p3, cnt, w, b.reshape(1, -1))
    return out.reshape(nchunk * n, f)


def _relu_comb_body(s3_ref, p3_ref, cnt_ref, b_ref, o_ref, *, nchunk, f):
    d = lax.rsqrt(cnt_ref[...][:, :1] + 1.0)
    h = jnp.concatenate([s3_ref[kk] + p3_ref[kk] for kk in range(nchunk)], axis=1)
    o_ref[...] = jax.nn.relu(h * d + b_ref[...])


def _relu_comb(s2d, p2d, cnt, b, nchunk, f, bm=1000):
    n = cnt.shape[0]
    s3 = s2d.reshape(nchunk, n, f)
    p3 = p2d.reshape(nchunk, n, f)
    body = functools.partial(_relu_comb_body, nchunk=nchunk, f=f)
    return pl.pallas_call(
        body,
        grid=(n // bm,),
        in_specs=[
            pl.BlockSpec((nchunk, bm, f), lambda i: (0, i, 0)),
            pl.BlockSpec((nchunk, bm, f), lambda i: (0, i, 0)),
            pl.BlockSpec((bm, 16), lambda i: (i, 0)),
            pl.BlockSpec((1, nchunk * f), lambda i: (0, 0)),
        ],
        out_specs=pl.BlockSpec((bm, nchunk * f), lambda i: (i, 0)),
        out_shape=jax.ShapeDtypeStruct((n, nchunk * f), F32),
    )(s3, p3, cnt, b.reshape(1, -1))


def _elu(x):
    return jnp.where(x > 0, x, jnp.exp(x) - 1.0)


def _contrast_body(za_ref, sums_ref, cnts_ref, p1_ref, pb1_ref, p2_ref, pb2_ref,
                   wq_ref, bq_ref, wk_ref, bk_ref, wv_ref, bv_ref, o_ref):
    g = za_ref.shape[0]
    g_up = sums_ref.shape[0] // 2
    sums = sums_ref[...]
    cnts = cnts_ref[...]
    cnt = jnp.clip(cnts[:g, :1] + cnts[g_up:g_up + g, :1], 1.0, None)
    zb = (sums[:g] + sums[g_up:g_up + g]) / cnt
    za = za_ref[...]
    p1 = p1_ref[...]
    p2 = p2_ref[...]
    za_p = jnp.dot(_elu(jnp.dot(za, p1, preferred_element_type=F32) + pb1_ref[...]),
                   p2, preferred_element_type=F32) + pb2_ref[...]
    zb_p = jnp.dot(_elu(jnp.dot(zb, p1, preferred_element_type=F32) + pb1_ref[...]),
                   p2, preferred_element_type=F32) + pb2_ref[...]
    scale = jnp.sqrt(za_p.shape[1] / 2.0).astype(F32)

    def ca(q_in, k_in, v_in):
        q = jnp.dot(q_in, wq_ref[...], preferred_element_type=F32) + bq_ref[...]
        k = jnp.dot(k_in, wk_ref[...], preferred_element_type=F32) + bk_ref[...]
        v = jnp.dot(v_in, wv_ref[...], preferred_element_type=F32) + bv_ref[...]
        logits = lax.dot_general(q, k, (((1,), (1,)), ((), ())),
                                 preferred_element_type=F32) / scale
        logits = logits - jnp.max(logits, axis=-1, keepdims=True)
        e = jnp.exp(logits)
        a = e / jnp.sum(e, axis=-1, keepdims=True)
        return jnp.dot(a, v, preferred_element_type=F32)

    o_ref[...] = jnp.concatenate([ca(za_p, zb_p, zb_p), ca(zb_p, za_p, za_p)], axis=1)


def _contrast_tc(za, sums, cnts, p, pre):
    m = za.shape[0]
    args = (za, sums, cnts, p[pre + 'P1'], p[pre + 'pb1'].reshape(1, -1),
            p[pre + 'P2'], p[pre + 'pb2'].reshape(1, -1),
            p[pre + 'Wq'], p[pre + 'bq'].reshape(1, -1),
            p[pre + 'Wk'], p[pre + 'bk'].reshape(1, -1),
            p[pre + 'Wv'], p[pre + 'bv'].reshape(1, -1))
    return pl.pallas_call(
        _contrast_body,
        out_shape=jax.ShapeDtypeStruct((m, 256), F32),
    )(*args)


def _pad_edges(edge_index, n):
    e = edge_index.shape[1]
    ept = -(-e // (_NS * 128 * _KG)) * 128 * _KG
    e_pad = _NS * ept
    pad = e_pad - e
    row = jnp.concatenate([edge_index[0],
                           jnp.arange(pad, dtype=I32) % jnp.int32(n)])
    col = jnp.concatenate([edge_index[1], jnp.full((pad,), n, I32)])
    return row.reshape(-1, _KG, 128), col.reshape(-1, _KG, 128)


def _pad_batch(batch, n, g, rpw):
    npad = 32 * rpw - n
    return jnp.concatenate([batch, jnp.full((npad,), g, I32)]
                           ).reshape(32, rpw // 128, 128)


def _graph_path(x, edge_index, batch, cnt, w0, b0, w1, b1, g, n, f, nchunk,
                rpw, w_partial, psize, zeros512f, zeros_g, zeros_c, ones128):
    row2, col2 = _pad_edges(edge_index, n)
    p1 = _node_mm(x, w0, cnt, nchunk, f)
    s1 = _spmm_sc(p1, row2, col2, zeros512f, n, f, nchunk)
    p2 = _layer2_mm(s1, p1, cnt, w1, b0, nchunk, f)
    s2 = _spmm_sc(p2, row2, col2, zeros512f, n, f, nchunk)
    h2 = _relu_comb(s2, p2, cnt, b1, nchunk, f)
    batch2 = _pad_batch(batch, n, g, rpw)
    sums, cnts = _gmp_sc(h2, batch2, zeros_g, zeros_c, ones128,
                         n, g, rpw, w_partial, psize)
    return sums, cnts


def kernel(aff_x, aff_adj, drug_x, drug_edge_index, drug_batch,
           target_x, target_edge_index, target_batch, params, num_drug, num_target):
    nd_g, nt_g = 1000, 1500
    n_d, n_t = drug_x.shape[0], target_x.shape[0]
    p = params

    ones128 = jnp.ones((128, 16), F32)
    zeros512_16 = jnp.zeros((512, 16), F32)
    zeros512_32 = jnp.zeros((512, 32), F32)
    zeros_g = jnp.zeros((96, 256), F32)
    zeros_c = jnp.zeros((96, 16), F32)

    aff_emb = _dense_gcn_tc(aff_x, aff_adj, p)

    row2d, col2d = _pad_edges(drug_edge_index, n_d)
    row2t, col2t = _pad_edges(target_edge_index, n_t)
    cnt_d, cnt_t = _deg_sc(col2d, col2t, ones128, zeros512_16, n_d, n_t)

    sums_d, cnts_d = _graph_path(
        drug_x, drug_edge_index, drug_batch, cnt_d,
        p['d_W0'], p['d_b0'], p['d_W1'], p['d_b1'], nd_g, n_d, 32, 8,
        1664, 30, 80, zeros512_32, zeros_g, zeros_c, ones128)
    sums_t, cnts_t = _graph_path(
        target_x, target_edge_index, target_batch, cnt_t,
        p['t_W0'], p['t_b0'], p['t_W1'], p['t_b1'], nt_g, n_t, 16, 16,
        3200, 31, 32, zeros512_16, zeros_g, zeros_c, ones128)

    drug_out = _contrast_tc(aff_emb[:nd_g], sums_d, cnts_d, p, 'dc_')
    target_out = _contrast_tc(aff_emb[nd_g:], sums_t, cnts_t, p, 'tc_')
    drug_out = drug_out + jnp.asarray(num_drug - nd_g, dtype=drug_out.dtype)
    target_out = target_out + jnp.asarray(num_target - nt_g, dtype=target_out.dtype)
    return (drug_out, target_out)


# speedup vs baseline: 3.0033x; 1.0545x over previous
import functools

import jax
import jax.numpy as jnp
from jax import lax
from jax.experimental import pallas as pl
from jax.experimental.pallas import tpu as pltpu
from jax.experimental.pallas import tpu_sc as plsc

F32 = jnp.float32
I32 = jnp.int32
_NC = 2
_NS = 16
_IW = 512


def _sc_mesh():
    return plsc.VectorSubcoreMesh(core_axis_name="c", subcore_axis_name="s",
                                  num_cores=_NC, num_subcores=_NS)


def _rpt8(n):
    return -(-(-(-(n + 16) // 16)) // 8) * 8


def _deg_sc(col3_d, col3_t, ones128, zeros512, nd, nt):
    rpt_d = _rpt8(nd)
    rpt_t = _rpt8(nt)
    n_alloc = 16 * max(rpt_d, rpt_t)
    gd = col3_d.shape[0] // _NS
    gt = col3_t.shape[0] // _NS

    @functools.partial(
        pl.kernel, mesh=_sc_mesh(),
        compiler_params=pltpu.CompilerParams(use_tc_tiling_on_sc=False),
        out_type=[jax.ShapeDtypeStruct((nd, 16), F32),
                  jax.ShapeDtypeStruct((nt, 16), F32)],
        scratch_types=[
            pltpu.VMEM((_IW,), I32),
            pltpu.VMEM((_IW, 16), F32),
            pltpu.VMEM_SHARED((n_alloc, 16), F32),
        ],
    )
    def k(cd_hbm, ct_hbm, ones_hbm, z_hbm, dd_hbm, dt_hbm, idx_v, ones_v, acc):
        c = lax.axis_index("c")
        s = lax.axis_index("s")
        pltpu.sync_copy(ones_hbm, ones_v)
        for n, rpt, col_hbm, ngrp, out_hbm, core in (
                (nd, rpt_d, cd_hbm, gd, dd_hbm, 0),
                (nt, rpt_t, ct_hbm, gt, dt_hbm, 1)):
            @pl.when(c == core)
            def _():
                row0 = s * rpt
                zfull, ztail = rpt // 512, rpt % 512
                for zi in range(zfull):
                    pltpu.sync_copy(z_hbm,
                                    acc.at[pl.ds(row0 + zi * 512, 512)])
                if ztail:
                    pltpu.sync_copy(z_hbm.at[pl.ds(0, ztail)],
                                    acc.at[pl.ds(row0 + zfull * 512, ztail)])
                plsc.subcore_barrier()

                def grp(g, carry):
                    pltpu.sync_copy(col_hbm.at[s * ngrp + g], idx_v)
                    pltpu.sync_copy(ones_v, acc.at[idx_v], add=True)
                    return carry
                lax.fori_loop(0, ngrp, grp, 0)
                plsc.subcore_barrier()

                last = n - 15 * rpt
                @pl.when(s < _NS - 1)
                def _():
                    pltpu.sync_copy(acc.at[pl.ds(row0, rpt)],
                                    out_hbm.at[pl.ds(row0, rpt)])
                @pl.when(s == _NS - 1)
                def _():
                    pltpu.sync_copy(acc.at[pl.ds(row0, last)],
                                    out_hbm.at[pl.ds(row0, last)])
                plsc.subcore_barrier()

    return k(col3_d, col3_t, ones128, zeros512)


def _spmm_sc(p2d, row3, col3, zeros512, n, f, nchunk):
    rpt = _rpt8(n)
    n_alloc = 16 * rpt
    ngrp = row3.shape[0] // _NS
    npass = nchunk // _NC

    @functools.partial(
        pl.kernel, mesh=_sc_mesh(),
        compiler_params=pltpu.CompilerParams(use_tc_tiling_on_sc=False),
        out_type=jax.ShapeDtypeStruct((nchunk * n, f), F32),
        scratch_types=[
            pltpu.VMEM((_IW,), I32),
            pltpu.VMEM((_IW,), I32),
            pltpu.VMEM((_IW, f), F32),
            pltpu.VMEM_SHARED((n_alloc, f), F32),
            pltpu.SemaphoreType.DMA,
        ],
    )
    def k(p_hbm, row_hbm, col_hbm, z_hbm, s_hbm, idxr, idxc, buf, acc, sem):
        c = lax.axis_index("c")
        s = lax.axis_index("s")
        row0 = s * rpt
        zfull, ztail = rpt // 512, rpt % 512
        last = n - 15 * rpt

        def one_pass(ppass, carry):
            chunk = ppass * _NC + c
            off = chunk * n
            for zi in range(zfull):
                pltpu.sync_copy(z_hbm, acc.at[pl.ds(row0 + zi * 512, 512)])
            if ztail:
                pltpu.sync_copy(z_hbm.at[pl.ds(0, ztail)],
                                acc.at[pl.ds(row0 + zfull * 512, ztail)])
            plsc.subcore_barrier()

            def grp(g, carry2):
                base = s * ngrp + g
                pltpu.sync_copy(row_hbm.at[base], idxr)
                pltpu.sync_copy(col_hbm.at[base], idxc)
                offv = jnp.full((16,), off, I32)
                for jj in range(_IW // 16):
                    sl = pl.ds(jj * 16, 16)
                    idxr[sl] = idxr[sl] + offv
                pltpu.async_copy(p_hbm.at[idxr], buf, sem).wait()
                pltpu.sync_copy(buf, acc.at[idxc], add=True)
                return carry2
            lax.fori_loop(0, ngrp, grp, 0)
            plsc.subcore_barrier()

            @pl.when(s < _NS - 1)
            def _():
                pltpu.sync_copy(acc.at[pl.ds(row0, rpt)],
                                s_hbm.at[pl.ds(off + row0, rpt)])
            @pl.when(s == _NS - 1)
            def _():
                pltpu.sync_copy(acc.at[pl.ds(row0, last)],
                                s_hbm.at[pl.ds(off + row0, last)])
            plsc.subcore_barrier()
            return carry

        lax.fori_loop(0, npass, one_pass, 0)

    return k(p2d, row3, col3, zeros512)


def _gmp_sc(h, batch3, zeros_g, zeros_c, ones128, n, g, rpw, w_partial, psize):
    g_pt = _rpt8(g)
    g_alloc = 16 * g_pt
    g_up = -(-g // 8) * 8
    nf_max = rpw // 128

    @functools.partial(
        pl.kernel, mesh=_sc_mesh(),
        compiler_params=pltpu.CompilerParams(use_tc_tiling_on_sc=False),
        out_type=[jax.ShapeDtypeStruct((_NC * g_up, 256), F32),
                  jax.ShapeDtypeStruct((_NC * g_up, 16), F32)],
        scratch_types=[
            pltpu.VMEM((128, 256), F32),
            pltpu.VMEM((128, 16), F32),
            pltpu.VMEM((nf_max, 128), I32),
            pltpu.VMEM_SHARED((g_alloc, 256), F32),
            pltpu.VMEM_SHARED((g_alloc, 16), F32),
        ],
    )
    def k(h_hbm, b_hbm, zg_hbm, zc_hbm, ones_hbm, s_hbm, c_hbm,
          buf, ones_v, idxb, accs, accc):
        c = lax.axis_index("c")
        s = lax.axis_index("s")
        w = c * _NS + s
        pltpu.sync_copy(ones_hbm, ones_v)
        grow0 = s * g_pt
        pltpu.sync_copy(zg_hbm.at[pl.ds(0, g_pt)], accs.at[pl.ds(grow0, g_pt)])
        pltpu.sync_copy(zc_hbm.at[pl.ds(0, g_pt)], accc.at[pl.ds(grow0, g_pt)])
        plsc.subcore_barrier()

        base = w * rpw
        nfull = jnp.clip((n - base) // 128, 0, nf_max)
        pltpu.sync_copy(b_hbm.at[w], idxb)

        def chunk(j, carry):
            r0 = base + j * 128
            pltpu.sync_copy(h_hbm.at[pl.ds(r0, 128)], buf)
            pltpu.sync_copy(buf, accs.at[idxb.at[j]], add=True)
            pltpu.sync_copy(ones_v, accc.at[idxb.at[j]], add=True)
            return carry
        lax.fori_loop(0, nfull, chunk, 0)

        if psize:
            p_j = (n - w_partial * rpw) // 128

            @pl.when(w == w_partial)
            def _():
                r0 = w_partial * rpw + p_j * 128
                pltpu.sync_copy(h_hbm.at[pl.ds(r0, psize)],
                                buf.at[pl.ds(0, psize)])
                pltpu.sync_copy(buf, accs.at[idxb.at[p_j]], add=True)
                pltpu.sync_copy(ones_v, accc.at[idxb.at[p_j]], add=True)
        plsc.subcore_barrier()

        glast = g - 15 * g_pt
        @pl.when(s < _NS - 1)
        def _():
            pltpu.sync_copy(accs.at[pl.ds(grow0, g_pt)],
                            s_hbm.at[pl.ds(c * g_up + grow0, g_pt)])
            pltpu.sync_copy(accc.at[pl.ds(grow0, g_pt)],
                            c_hbm.at[pl.ds(c * g_up + grow0, g_pt)])
        @pl.when(s == _NS - 1)
        def _():
            pltpu.sync_copy(accs.at[pl.ds(grow0, glast)],
                            s_hbm.at[pl.ds(c * g_up + grow0, glast)])
            pltpu.sync_copy(accc.at[pl.ds(grow0, glast)],
                            c_hbm.at[pl.ds(c * g_up + grow0, glast)])

    return k(h, batch3, zeros_g, zeros_c, ones128)


def _rowsum_body(a_ref, o_ref):
    o_ref[...] = jnp.broadcast_to(jnp.sum(a_ref[...], axis=1, keepdims=True),
                                  o_ref.shape)


def _aff_rowsum(a_pad):
    m = a_pad.shape[0]
    bm = 256
    return pl.pallas_call(
        _rowsum_body,
        grid=(m // bm,),
        in_specs=[pl.BlockSpec((bm, m), lambda i: (i, 0))],
        out_specs=pl.BlockSpec((bm, 8), lambda i: (i, 0)),
        out_shape=jax.ShapeDtypeStruct((m, 8), F32),
    )(a_pad)


def _mm_scale_body(x_ref, w_ref, s_ref, o_ref):
    d = lax.rsqrt(jnp.clip(s_ref[...][:, :1], 1.0, None))
    o_ref[...] = jnp.dot(x_ref[...], w_ref[...], preferred_element_type=F32) * d


def _mm_scale(x, w, s, bm=256):
    m, kk = x.shape
    n = w.shape[1]
    return pl.pallas_call(
        _mm_scale_body,
        grid=(m // bm,),
        in_specs=[
            pl.BlockSpec((bm, kk), lambda i: (i, 0)),
            pl.BlockSpec((kk, n), lambda i: (0, 0)),
            pl.BlockSpec((bm, 8), lambda i: (i, 0)),
        ],
        out_specs=pl.BlockSpec((bm, n), lambda i: (i, 0)),
        out_shape=jax.ShapeDtypeStruct((m, n), F32),
    )(x, w, s)


def _amm_body(a_ref, y_ref, s_ref, b_ref, o_ref):
    t = jnp.dot(a_ref[...], y_ref[...], preferred_element_type=F32)
    d = lax.rsqrt(jnp.clip(s_ref[...][:, :1], 1.0, None))
    o_ref[...] = jax.nn.relu(t * d + b_ref[...])


def _amm(a, y, s, b, bm=256):
    m = a.shape[0]
    kk, n = y.shape
    return pl.pallas_call(
        _amm_body,
        grid=(m // bm,),
        in_specs=[
            pl.BlockSpec((bm, kk), lambda i: (i, 0)),
            pl.BlockSpec((kk, n), lambda i: (0, 0)),
            pl.BlockSpec((bm, 8), lambda i: (i, 0)),
            pl.BlockSpec((1, n), lambda i: (0, 0)),
        ],
        out_specs=pl.BlockSpec((bm, n), lambda i: (i, 0)),
        out_shape=jax.ShapeDtypeStruct((m, n), F32),
    )(a, y, s, b)


def _dense_gcn_tc(aff_x, aff_adj, p):
    n = aff_adj.shape[0]
    m = ((n + 255) // 256) * 256
    a_pad = jnp.pad(aff_adj, ((0, m - n), (0, m - n)))
    x_pad = jnp.pad(aff_x, ((0, m - n), (0, 0)))
    s = _aff_rowsum(a_pad)
    y = _mm_scale(x_pad, p['aff_W0'], s)
    h1 = _amm(a_pad, y, s, p['aff_b0'].reshape(1, -1))
    y2 = _mm_scale(h1, p['aff_W1'], s)
    h2 = _amm(a_pad, y2, s, p['aff_b1'].reshape(1, -1))
    return h2[:n]


def _node_mm_body(x_ref, w_ref, cnt_ref, o_ref, *, nchunk, f):
    d = lax.rsqrt(cnt_ref[...][:, :1] + 1.0)
    t = jnp.dot(x_ref[...], w_ref[...], preferred_element_type=F32) * d
    for kk in range(nchunk):
        o_ref[kk] = t[:, kk * f:(kk + 1) * f]


def _node_mm(x, w, cnt, nchunk, f, bm=1000):
    n, kdim = x.shape
    body = functools.partial(_node_mm_body, nchunk=nchunk, f=f)
    out = pl.pallas_call(
        body,
        grid=(n // bm,),
        in_specs=[
            pl.BlockSpec((bm, kdim), lambda i: (i, 0)),
            pl.BlockSpec((kdim, w.shape[1]), lambda i: (0, 0)),
            pl.BlockSpec((bm, 16), lambda i: (i, 0)),
        ],
        out_specs=pl.BlockSpec((nchunk, bm, f), lambda i: (0, i, 0)),
        out_shape=jax.ShapeDtypeStruct((nchunk, n, f), F32),
    )(x, w, cnt)
    return out.reshape(nchunk * n, f)


def _layer2_body(s3_ref, p3_ref, cnt_ref, w_ref, b_ref, o_ref, *, nchunk, f):
    d = lax.rsqrt(cnt_ref[...][:, :1] + 1.0)
    h = jnp.concatenate([s3_ref[kk] + p3_ref[kk] for kk in range(nchunk)], axis=1)
    h = jax.nn.relu(h * d + b_ref[...])
    t = jnp.dot(h, w_ref[...], preferred_element_type=F32) * d
    for kk in range(nchunk):
        o_ref[kk] = t[:, kk * f:(kk + 1) * f]


def _layer2_mm(s2d, p2d, cnt, w, b, nchunk, f, bm=1000):
    n = cnt.shape[0]
    s3 = s2d.reshape(nchunk, n, f)
    p3 = p2d.reshape(nchunk, n, f)
    body = functools.partial(_layer2_body, nchunk=nchunk, f=f)
    out = pl.pallas_call(
        body,
        grid=(n // bm,),
        in_specs=[
            pl.BlockSpec((nchunk, bm, f), lambda i: (0, i, 0)),
            pl.BlockSpec((nchunk, bm, f), lambda i: (0, i, 0)),
            pl.BlockSpec((bm, 16), lambda i: (i, 0)),
            pl.BlockSpec((w.shape[0], w.shape[1]), lambda i: (0, 0)),
            pl.BlockSpec((1, w.shape[1]), lambda i: (0, 0)),
        ],
        out_specs=pl.BlockSpec((nchunk, bm, f), lambda i: (0, i, 0)),
        out_shape=jax.ShapeDtypeStruct((nchunk, n, f), F32),
    )(s3, p3, cnt, w, b.reshape(1, -1))
    return out.reshape(nchunk * n, f)


def _relu_comb_body(s3_ref, p3_ref, cnt_ref, b_ref, o_ref, *, nchunk, f):
    d = lax.rsqrt(cnt_ref[...][:, :1] + 1.0)
    h = jnp.concatenate([s3_ref[kk] + p3_ref[kk] for kk in range(nchunk)], axis=1)
    o_ref[...] = jax.nn.relu(h * d + b_ref[...])


def _relu_comb(s2d, p2d, cnt, b, nchunk, f, bm=1000):
    n = cnt.shape[0]
    s3 = s2d.reshape(nchunk, n, f)
    p3 = p2d.reshape(nchunk, n, f)
    body = functools.partial(_relu_comb_body, nchunk=nchunk, f=f)
    return pl.pallas_call(
        body,
        grid=(n // bm,),
        in_specs=[
            pl.BlockSpec((nchunk, bm, f), lambda i: (0, i, 0)),
            pl.BlockSpec((nchunk, bm, f), lambda i: (0, i, 0)),
            pl.BlockSpec((bm, 16), lambda i: (i, 0)),
            pl.BlockSpec((1, nchunk * f), lambda i: (0, 0)),
        ],
        out_specs=pl.BlockSpec((bm, nchunk * f), lambda i: (i, 0)),
        out_shape=jax.ShapeDtypeStruct((n, nchunk * f), F32),
    )(s3, p3, cnt, b.reshape(1, -1))


def _elu(x):
    return jnp.where(x > 0, x, jnp.exp(x) - 1.0)


def _contrast_body(za_ref, sums_ref, cnts_ref, p1_ref, pb1_ref, p2_ref, pb2_ref,
                   wq_ref, bq_ref, wk_ref, bk_ref, wv_ref, bv_ref, o_ref):
    g = za_ref.shape[0]
    g_up = sums_ref.shape[0] // 2
    sums = sums_ref[...]
    cnts = cnts_ref[...]
    cnt = jnp.clip(cnts[:g, :1] + cnts[g_up:g_up + g, :1], 1.0, None)
    zb = (sums[:g] + sums[g_up:g_up + g]) / cnt
    za = za_ref[...]
    p1 = p1_ref[...]
    p2 = p2_ref[...]
    za_p = jnp.dot(_elu(jnp.dot(za, p1, preferred_element_type=F32) + pb1_ref[...]),
                   p2, preferred_element_type=F32) + pb2_ref[...]
    zb_p = jnp.dot(_elu(jnp.dot(zb, p1, preferred_element_type=F32) + pb1_ref[...]),
                   p2, preferred_element_type=F32) + pb2_ref[...]
    scale = jnp.sqrt(za_p.shape[1] / 2.0).astype(F32)

    def ca(q_in, k_in, v_in):
        q = jnp.dot(q_in, wq_ref[...], preferred_element_type=F32) + bq_ref[...]
        k = jnp.dot(k_in, wk_ref[...], preferred_element_type=F32) + bk_ref[...]
        v = jnp.dot(v_in, wv_ref[...], preferred_element_type=F32) + bv_ref[...]
        logits = lax.dot_general(q, k, (((1,), (1,)), ((), ())),
                                 preferred_element_type=F32) / scale
        logits = logits - jnp.max(logits, axis=-1, keepdims=True)
        e = jnp.exp(logits)
        a = e / jnp.sum(e, axis=-1, keepdims=True)
        return jnp.dot(a, v, preferred_element_type=F32)

    o_ref[...] = jnp.concatenate([ca(za_p, zb_p, zb_p), ca(zb_p, za_p, za_p)], axis=1)


def _contrast_tc(za, sums, cnts, p, pre):
    m = za.shape[0]
    args = (za, sums, cnts, p[pre + 'P1'], p[pre + 'pb1'].reshape(1, -1),
            p[pre + 'P2'], p[pre + 'pb2'].reshape(1, -1),
            p[pre + 'Wq'], p[pre + 'bq'].reshape(1, -1),
            p[pre + 'Wk'], p[pre + 'bk'].reshape(1, -1),
            p[pre + 'Wv'], p[pre + 'bv'].reshape(1, -1))
    return pl.pallas_call(
        _contrast_body,
        out_shape=jax.ShapeDtypeStruct((m, 256), F32),
    )(*args)


def _pad_edges(edge_index, n):
    e = edge_index.shape[1]
    ept = -(-e // (_NS * _IW)) * _IW
    e_pad = _NS * ept
    pad = e_pad - e
    row = jnp.concatenate([edge_index[0],
                           jnp.arange(pad, dtype=I32) % jnp.int32(n)])
    col = jnp.concatenate([edge_index[1], jnp.full((pad,), n, I32)])
    return row.reshape(-1, _IW), col.reshape(-1, _IW)


def _pad_batch(batch, n, g, rpw):
    npad = 32 * rpw - n
    return jnp.concatenate([batch, jnp.full((npad,), g, I32)]
                           ).reshape(32, rpw // 128, 128)


def _graph_path(x, edge_index, batch, cnt, w0, b0, w1, b1, g, n, f, nchunk,
                rpw, w_partial, psize, zeros512f, zeros_g, zeros_c, ones128):
    row2, col2 = _pad_edges(edge_index, n)
    p1 = _node_mm(x, w0, cnt, nchunk, f)
    s1 = _spmm_sc(p1, row2, col2, zeros512f, n, f, nchunk)
    p2 = _layer2_mm(s1, p1, cnt, w1, b0, nchunk, f)
    s2 = _spmm_sc(p2, row2, col2, zeros512f, n, f, nchunk)
    h2 = _relu_comb(s2, p2, cnt, b1, nchunk, f)
    batch2 = _pad_batch(batch, n, g, rpw)
    sums, cnts = _gmp_sc(h2, batch2, zeros_g, zeros_c, ones128,
                         n, g, rpw, w_partial, psize)
    return sums, cnts


def kernel(aff_x, aff_adj, drug_x, drug_edge_index, drug_batch,
           target_x, target_edge_index, target_batch, params, num_drug, num_target):
    nd_g, nt_g = 1000, 1500
    n_d, n_t = drug_x.shape[0], target_x.shape[0]
    p = params

    ones128 = jnp.ones((128, 16), F32)
    ones512 = jnp.ones((_IW, 16), F32)
    zeros512_16 = jnp.zeros((512, 16), F32)
    zeros512_32 = jnp.zeros((512, 32), F32)
    zeros_g = jnp.zeros((96, 256), F32)
    zeros_c = jnp.zeros((96, 16), F32)

    aff_emb = _dense_gcn_tc(aff_x, aff_adj, p)

    row2d, col2d = _pad_edges(drug_edge_index, n_d)
    row2t, col2t = _pad_edges(target_edge_index, n_t)
    cnt_d, cnt_t = _deg_sc(col2d, col2t, ones512, zeros512_16, n_d, n_t)

    sums_d, cnts_d = _graph_path(
        drug_x, drug_edge_index, drug_batch, cnt_d,
        p['d_W0'], p['d_b0'], p['d_W1'], p['d_b1'], nd_g, n_d, 32, 8,
        1664, 30, 80, zeros512_32, zeros_g, zeros_c, ones128)
    sums_t, cnts_t = _graph_path(
        target_x, target_edge_index, target_batch, cnt_t,
        p['t_W0'], p['t_b0'], p['t_W1'], p['t_b1'], nt_g, n_t, 16, 16,
        3200, 31, 32, zeros512_16, zeros_g, zeros_c, ones128)

    drug_out = _contrast_tc(aff_emb[:nd_g], sums_d, cnts_d, p, 'dc_')
    target_out = _contrast_tc(aff_emb[nd_g:], sums_t, cnts_t, p, 'tc_')
    drug_out = drug_out + jnp.asarray(num_drug - nd_g, dtype=drug_out.dtype)
    target_out = target_out + jnp.asarray(num_target - nt_g, dtype=target_out.dtype)
    return (drug_out, target_out)


# speedup vs baseline: 4.9412x; 1.6452x over previous
import functools

import jax
import jax.numpy as jnp
from jax import lax
from jax.experimental import pallas as pl
from jax.experimental.pallas import tpu as pltpu
from jax.experimental.pallas import tpu_sc as plsc

F32 = jnp.float32
BF16 = jnp.bfloat16
I32 = jnp.int32
_NC = 2
_NS = 16
_IW = 512


def _sc_mesh():
    return plsc.VectorSubcoreMesh(core_axis_name="c", subcore_axis_name="s",
                                  num_cores=_NC, num_subcores=_NS)


def _rpt8(n):
    return -(-(-(-(n + 16) // 16)) // 8) * 8


def _deg_sc(col3_d, col3_t, ones128, zeros512, nd, nt):
    rpt_d = _rpt8(nd)
    rpt_t = _rpt8(nt)
    n_alloc = 16 * max(rpt_d, rpt_t)
    gd = col3_d.shape[0] // _NS
    gt = col3_t.shape[0] // _NS

    @functools.partial(
        pl.kernel, mesh=_sc_mesh(),
        compiler_params=pltpu.CompilerParams(use_tc_tiling_on_sc=False),
        out_type=[jax.ShapeDtypeStruct((nd, 16), F32),
                  jax.ShapeDtypeStruct((nt, 16), F32)],
        scratch_types=[
            pltpu.VMEM((_IW,), I32),
            pltpu.VMEM((_IW, 16), F32),
            pltpu.VMEM_SHARED((n_alloc, 16), F32),
        ],
    )
    def k(cd_hbm, ct_hbm, ones_hbm, z_hbm, dd_hbm, dt_hbm, idx_v, ones_v, acc):
        c = lax.axis_index("c")
        s = lax.axis_index("s")
        pltpu.sync_copy(ones_hbm, ones_v)
        for n, rpt, col_hbm, ngrp, out_hbm, core in (
                (nd, rpt_d, cd_hbm, gd, dd_hbm, 0),
                (nt, rpt_t, ct_hbm, gt, dt_hbm, 1)):
            @pl.when(c == core)
            def _():
                row0 = s * rpt
                zfull, ztail = rpt // 512, rpt % 512
                for zi in range(zfull):
                    pltpu.sync_copy(z_hbm,
                                    acc.at[pl.ds(row0 + zi * 512, 512)])
                if ztail:
                    pltpu.sync_copy(z_hbm.at[pl.ds(0, ztail)],
                                    acc.at[pl.ds(row0 + zfull * 512, ztail)])
                plsc.subcore_barrier()

                def grp(g, carry):
                    pltpu.sync_copy(col_hbm.at[s * ngrp + g], idx_v)
                    pltpu.sync_copy(ones_v, acc.at[idx_v], add=True)
                    return carry
                lax.fori_loop(0, ngrp, grp, 0)
                plsc.subcore_barrier()

                last = n - 15 * rpt
                @pl.when(s < _NS - 1)
                def _():
                    pltpu.sync_copy(acc.at[pl.ds(row0, rpt)],
                                    out_hbm.at[pl.ds(row0, rpt)])
                @pl.when(s == _NS - 1)
                def _():
                    pltpu.sync_copy(acc.at[pl.ds(row0, last)],
                                    out_hbm.at[pl.ds(row0, last)])
                plsc.subcore_barrier()

    return k(col3_d, col3_t, ones128, zeros512)


def _spmm_sc(p2d, row3, col3, zeros512, n, f, nchunk):
    rpt = _rpt8(n)
    n_alloc = 16 * rpt
    ngrp = row3.shape[0] // _NS
    npass = nchunk // _NC

    @functools.partial(
        pl.kernel, mesh=_sc_mesh(),
        compiler_params=pltpu.CompilerParams(use_tc_tiling_on_sc=False),
        out_type=jax.ShapeDtypeStruct((nchunk * n, f), BF16),
        scratch_types=[
            pltpu.VMEM((_IW,), I32),
            pltpu.VMEM((_IW,), I32),
            pltpu.VMEM((_IW, f), BF16),
            pltpu.VMEM_SHARED((n_alloc, f), BF16),
            pltpu.SemaphoreType.DMA,
        ],
    )
    def k(p_hbm, row_hbm, col_hbm, z_hbm, s_hbm, idxr, idxc, buf, acc, sem):
        c = lax.axis_index("c")
        s = lax.axis_index("s")
        row0 = s * rpt
        zfull, ztail = rpt // 512, rpt % 512
        last = n - 15 * rpt

        def one_pass(ppass, carry):
            chunk = ppass * _NC + c
            off = chunk * n
            for zi in range(zfull):
                pltpu.sync_copy(z_hbm, acc.at[pl.ds(row0 + zi * 512, 512)])
            if ztail:
                pltpu.sync_copy(z_hbm.at[pl.ds(0, ztail)],
                                acc.at[pl.ds(row0 + zfull * 512, ztail)])
            plsc.subcore_barrier()

            def grp(g, carry2):
                base = s * ngrp + g
                pltpu.sync_copy(row_hbm.at[base], idxr)
                pltpu.sync_copy(col_hbm.at[base], idxc)
                offv = jnp.full((16,), off, I32)
                for jj in range(_IW // 16):
                    sl = pl.ds(jj * 16, 16)
                    idxr[sl] = idxr[sl] + offv
                pltpu.async_copy(p_hbm.at[idxr], buf, sem).wait()
                pltpu.sync_copy(buf, acc.at[idxc], add=True)
                return carry2
            lax.fori_loop(0, ngrp, grp, 0)
            plsc.subcore_barrier()

            @pl.when(s < _NS - 1)
            def _():
                pltpu.sync_copy(acc.at[pl.ds(row0, rpt)],
                                s_hbm.at[pl.ds(off + row0, rpt)])
            @pl.when(s == _NS - 1)
            def _():
                pltpu.sync_copy(acc.at[pl.ds(row0, last)],
                                s_hbm.at[pl.ds(off + row0, last)])
            plsc.subcore_barrier()
            return carry

        lax.fori_loop(0, npass, one_pass, 0)

    return k(p2d, row3, col3, zeros512)


def _gmp_sc(h, batch3, zeros_g, zeros_c, ones128, n, g, rpw, w_partial, psize):
    g_pt = _rpt8(g)
    g_alloc = 16 * g_pt
    g_up = -(-g // 8) * 8
    nf_max = rpw // 128

    @functools.partial(
        pl.kernel, mesh=_sc_mesh(),
        compiler_params=pltpu.CompilerParams(use_tc_tiling_on_sc=False),
        out_type=[jax.ShapeDtypeStruct((_NC * g_up, 256), F32),
                  jax.ShapeDtypeStruct((_NC * g_up, 16), F32)],
        scratch_types=[
            pltpu.VMEM((128, 256), F32),
            pltpu.VMEM((128, 16), F32),
            pltpu.VMEM((nf_max, 128), I32),
            pltpu.VMEM_SHARED((g_alloc, 256), F32),
            pltpu.VMEM_SHARED((g_alloc, 16), F32),
        ],
    )
    def k(h_hbm, b_hbm, zg_hbm, zc_hbm, ones_hbm, s_hbm, c_hbm,
          buf, ones_v, idxb, accs, accc):
        c = lax.axis_index("c")
        s = lax.axis_index("s")
        w = c * _NS + s
        pltpu.sync_copy(ones_hbm, ones_v)
        grow0 = s * g_pt
        pltpu.sync_copy(zg_hbm.at[pl.ds(0, g_pt)], accs.at[pl.ds(grow0, g_pt)])
        pltpu.sync_copy(zc_hbm.at[pl.ds(0, g_pt)], accc.at[pl.ds(grow0, g_pt)])
        plsc.subcore_barrier()

        base = w * rpw
        nfull = jnp.clip((n - base) // 128, 0, nf_max)
        pltpu.sync_copy(b_hbm.at[w], idxb)

        def chunk(j, carry):
            r0 = base + j * 128
            pltpu.sync_copy(h_hbm.at[pl.ds(r0, 128)], buf)
            pltpu.sync_copy(buf, accs.at[idxb.at[j]], add=True)
            pltpu.sync_copy(ones_v, accc.at[idxb.at[j]], add=True)
            return carry
        lax.fori_loop(0, nfull, chunk, 0)

        if psize:
            p_j = (n - w_partial * rpw) // 128

            @pl.when(w == w_partial)
            def _():
                r0 = w_partial * rpw + p_j * 128
                pltpu.sync_copy(h_hbm.at[pl.ds(r0, psize)],
                                buf.at[pl.ds(0, psize)])
                pltpu.sync_copy(buf, accs.at[idxb.at[p_j]], add=True)
                pltpu.sync_copy(ones_v, accc.at[idxb.at[p_j]], add=True)
        plsc.subcore_barrier()

        glast = g - 15 * g_pt
        @pl.when(s < _NS - 1)
        def _():
            pltpu.sync_copy(accs.at[pl.ds(grow0, g_pt)],
                            s_hbm.at[pl.ds(c * g_up + grow0, g_pt)])
            pltpu.sync_copy(accc.at[pl.ds(grow0, g_pt)],
                            c_hbm.at[pl.ds(c * g_up + grow0, g_pt)])
        @pl.when(s == _NS - 1)
        def _():
            pltpu.sync_copy(accs.at[pl.ds(grow0, glast)],
                            s_hbm.at[pl.ds(c * g_up + grow0, glast)])
            pltpu.sync_copy(accc.at[pl.ds(grow0, glast)],
                            c_hbm.at[pl.ds(c * g_up + grow0, glast)])

    return k(h, batch3, zeros_g, zeros_c, ones128)


def _rowsum_body(a_ref, o_ref):
    o_ref[...] = jnp.broadcast_to(jnp.sum(a_ref[...], axis=1, keepdims=True),
                                  o_ref.shape)


def _aff_rowsum(a_pad):
    m = a_pad.shape[0]
    bm = 256
    return pl.pallas_call(
        _rowsum_body,
        grid=(m // bm,),
        in_specs=[pl.BlockSpec((bm, m), lambda i: (i, 0))],
        out_specs=pl.BlockSpec((bm, 8), lambda i: (i, 0)),
        out_shape=jax.ShapeDtypeStruct((m, 8), F32),
    )(a_pad)


def _mm_scale_body(x_ref, w_ref, s_ref, o_ref):
    d = lax.rsqrt(jnp.clip(s_ref[...][:, :1], 1.0, None))
    o_ref[...] = jnp.dot(x_ref[...], w_ref[...], preferred_element_type=F32) * d


def _mm_scale(x, w, s, bm=256):
    m, kk = x.shape
    n = w.shape[1]
    return pl.pallas_call(
        _mm_scale_body,
        grid=(m // bm,),
        in_specs=[
            pl.BlockSpec((bm, kk), lambda i: (i, 0)),
            pl.BlockSpec((kk, n), lambda i: (0, 0)),
            pl.BlockSpec((bm, 8), lambda i: (i, 0)),
        ],
        out_specs=pl.BlockSpec((bm, n), lambda i: (i, 0)),
        out_shape=jax.ShapeDtypeStruct((m, n), F32),
    )(x, w, s)


def _amm_body(a_ref, y_ref, s_ref, b_ref, o_ref):
    t = jnp.dot(a_ref[...], y_ref[...], preferred_element_type=F32)
    d = lax.rsqrt(jnp.clip(s_ref[...][:, :1], 1.0, None))
    o_ref[...] = jax.nn.relu(t * d + b_ref[...])


def _amm(a, y, s, b, bm=256):
    m = a.shape[0]
    kk, n = y.shape
    return pl.pallas_call(
        _amm_body,
        grid=(m // bm,),
        in_specs=[
            pl.BlockSpec((bm, kk), lambda i: (i, 0)),
            pl.BlockSpec((kk, n), lambda i: (0, 0)),
            pl.BlockSpec((bm, 8), lambda i: (i, 0)),
            pl.BlockSpec((1, n), lambda i: (0, 0)),
        ],
        out_specs=pl.BlockSpec((bm, n), lambda i: (i, 0)),
        out_shape=jax.ShapeDtypeStruct((m, n), F32),
    )(a, y, s, b)


def _dense_gcn_tc(aff_x, aff_adj, p):
    n = aff_adj.shape[0]
    m = ((n + 255) // 256) * 256
    a_pad = jnp.pad(aff_adj, ((0, m - n), (0, m - n)))
    x_pad = jnp.pad(aff_x, ((0, m - n), (0, 0)))
    s = _aff_rowsum(a_pad)
    y = _mm_scale(x_pad, p['aff_W0'], s)
    h1 = _amm(a_pad, y, s, p['aff_b0'].reshape(1, -1))
    y2 = _mm_scale(h1, p['aff_W1'], s)
    h2 = _amm(a_pad, y2, s, p['aff_b1'].reshape(1, -1))
    return h2[:n]


def _node_mm_body(x_ref, w_ref, cnt_ref, o_ref, *, nchunk, f):
    d = lax.rsqrt(cnt_ref[...][:, :1] + 1.0)
    t = jnp.dot(x_ref[...], w_ref[...], preferred_element_type=F32) * d
    for kk in range(nchunk):
        o_ref[kk] = t[:, kk * f:(kk + 1) * f].astype(BF16)


def _node_mm(x, w, cnt, nchunk, f, bm=2000):
    n, kdim = x.shape
    body = functools.partial(_node_mm_body, nchunk=nchunk, f=f)
    out = pl.pallas_call(
        body,
        grid=(n // bm,),
        in_specs=[
            pl.BlockSpec((bm, kdim), lambda i: (i, 0)),
            pl.BlockSpec((kdim, w.shape[1]), lambda i: (0, 0)),
            pl.BlockSpec((bm, 16), lambda i: (i, 0)),
        ],
        out_specs=pl.BlockSpec((nchunk, bm, f), lambda i: (0, i, 0)),
        out_shape=jax.ShapeDtypeStruct((nchunk, n, f), BF16),
    )(x, w, cnt)
    return out.reshape(nchunk * n, f)


def _layer2_body(s3_ref, p3_ref, cnt_ref, w_ref, b_ref, o_ref, *, nchunk, f):
    d = lax.rsqrt(cnt_ref[...][:, :1] + 1.0)
    h = jnp.concatenate(
        [s3_ref[kk].astype(F32) + p3_ref[kk].astype(F32) for kk in range(nchunk)],
        axis=1)
    h = jax.nn.relu(h * d + b_ref[...])
    t = jnp.dot(h, w_ref[...], preferred_element_type=F32) * d
    for kk in range(nchunk):
        o_ref[kk] = t[:, kk * f:(kk + 1) * f].astype(BF16)


def _layer2_mm(s2d, p2d, cnt, w, b, nchunk, f, bm=2000):
    n = cnt.shape[0]
    s3 = s2d.reshape(nchunk, n, f)
    p3 = p2d.reshape(nchunk, n, f)
    body = functools.partial(_layer2_body, nchunk=nchunk, f=f)
    out = pl.pallas_call(
        body,
        grid=(n // bm,),
        in_specs=[
            pl.BlockSpec((nchunk, bm, f), lambda i: (0, i, 0)),
            pl.BlockSpec((nchunk, bm, f), lambda i: (0, i, 0)),
            pl.BlockSpec((bm, 16), lambda i: (i, 0)),
            pl.BlockSpec((w.shape[0], w.shape[1]), lambda i: (0, 0)),
            pl.BlockSpec((1, w.shape[1]), lambda i: (0, 0)),
        ],
        out_specs=pl.BlockSpec((nchunk, bm, f), lambda i: (0, i, 0)),
        out_shape=jax.ShapeDtypeStruct((nchunk, n, f), BF16),
    )(s3, p3, cnt, w, b.reshape(1, -1))
    return out.reshape(nchunk * n, f)


def _relu_comb_body(s3_ref, p3_ref, cnt_ref, b_ref, o_ref, *, nchunk, f):
    d = lax.rsqrt(cnt_ref[...][:, :1] + 1.0)
    h = jnp.concatenate(
        [s3_ref[kk].astype(F32) + p3_ref[kk].astype(F32) for kk in range(nchunk)],
        axis=1)
    o_ref[...] = jax.nn.relu(h * d + b_ref[...])


def _relu_comb(s2d, p2d, cnt, b, nchunk, f, bm=2000):
    n = cnt.shape[0]
    s3 = s2d.reshape(nchunk, n, f)
    p3 = p2d.reshape(nchunk, n, f)
    body = functools.partial(_relu_comb_body, nchunk=nchunk, f=f)
    return pl.pallas_call(
        body,
        grid=(n // bm,),
        in_specs=[
            pl.BlockSpec((nchunk, bm, f), lambda i: (0, i, 0)),
            pl.BlockSpec((nchunk, bm, f), lambda i: (0, i, 0)),
            pl.BlockSpec((bm, 16), lambda i: (i, 0)),
            pl.BlockSpec((1, nchunk * f), lambda i: (0, 0)),
        ],
        out_specs=pl.BlockSpec((bm, nchunk * f), lambda i: (i, 0)),
        out_shape=jax.ShapeDtypeStruct((n, nchunk * f), F32),
    )(s3, p3, cnt, b.reshape(1, -1))


def _elu(x):
    return jnp.where(x > 0, x, jnp.exp(x) - 1.0)


def _contrast_body(za_ref, sums_ref, cnts_ref, p1_ref, pb1_ref, p2_ref, pb2_ref,
                   wq_ref, bq_ref, wk_ref, bk_ref, wv_ref, bv_ref, o_ref):
    g = za_ref.shape[0]
    g_up = sums_ref.shape[0] // 2
    sums = sums_ref[...]
    cnts = cnts_ref[...]
    cnt = jnp.clip(cnts[:g, :1] + cnts[g_up:g_up + g, :1], 1.0, None)
    zb = (sums[:g] + sums[g_up:g_up + g]) / cnt
    za = za_ref[...]
    p1 = p1_ref[...]
    p2 = p2_ref[...]
    za_p = jnp.dot(_elu(jnp.dot(za, p1, preferred_element_type=F32) + pb1_ref[...]),
                   p2, preferred_element_type=F32) + pb2_ref[...]
    zb_p = jnp.dot(_elu(jnp.dot(zb, p1, preferred_element_type=F32) + pb1_ref[...]),
                   p2, preferred_element_type=F32) + pb2_ref[...]
    scale = jnp.sqrt(za_p.shape[1] / 2.0).astype(F32)

    def ca(q_in, k_in, v_in):
        q = jnp.dot(q_in, wq_ref[...], preferred_element_type=F32) + bq_ref[...]
        k = jnp.dot(k_in, wk_ref[...], preferred_element_type=F32) + bk_ref[...]
        v = jnp.dot(v_in, wv_ref[...], preferred_element_type=F32) + bv_ref[...]
        logits = lax.dot_general(q, k, (((1,), (1,)), ((), ())),
                                 preferred_element_type=F32) / scale
        logits = logits - jnp.max(logits, axis=-1, keepdims=True)
        e = jnp.exp(logits)
        a = e / jnp.sum(e, axis=-1, keepdims=True)
        return jnp.dot(a, v, preferred_element_type=F32)

    o_ref[...] = jnp.concatenate([ca(za_p, zb_p, zb_p), ca(zb_p, za_p, za_p)], axis=1)


def _contrast_tc(za, sums, cnts, p, pre):
    m = za.shape[0]
    args = (za, sums, cnts, p[pre + 'P1'], p[pre + 'pb1'].reshape(1, -1),
            p[pre + 'P2'], p[pre + 'pb2'].reshape(1, -1),
            p[pre + 'Wq'], p[pre + 'bq'].reshape(1, -1),
            p[pre + 'Wk'], p[pre + 'bk'].reshape(1, -1),
            p[pre + 'Wv'], p[pre + 'bv'].reshape(1, -1))
    return pl.pallas_call(
        _contrast_body,
        out_shape=jax.ShapeDtypeStruct((m, 256), F32),
    )(*args)


def _pad_edges(edge_index, n):
    e = edge_index.shape[1]
    ept = -(-e // (_NS * _IW)) * _IW
    e_pad = _NS * ept
    pad = e_pad - e
    row = jnp.concatenate([edge_index[0],
                           jnp.arange(pad, dtype=I32) % jnp.int32(n)])
    col = jnp.concatenate([edge_index[1], jnp.full((pad,), n, I32)])
    return row.reshape(-1, _IW), col.reshape(-1, _IW)


def _pad_batch(batch, n, g, rpw):
    npad = 32 * rpw - n
    return jnp.concatenate([batch, jnp.full((npad,), g, I32)]
                           ).reshape(32, rpw // 128, 128)


def _graph_path(x, edge_index, batch, cnt, w0, b0, w1, b1, g, n, f, nchunk,
                rpw, w_partial, psize, zeros512f, zeros_g, zeros_c, ones128):
    row2, col2 = _pad_edges(edge_index, n)
    p1 = _node_mm(x, w0, cnt, nchunk, f)
    s1 = _spmm_sc(p1, row2, col2, zeros512f, n, f, nchunk)
    p2 = _layer2_mm(s1, p1, cnt, w1, b0, nchunk, f)
    s2 = _spmm_sc(p2, row2, col2, zeros512f, n, f, nchunk)
    h2 = _relu_comb(s2, p2, cnt, b1, nchunk, f)
    batch2 = _pad_batch(batch, n, g, rpw)
    sums, cnts = _gmp_sc(h2, batch2, zeros_g, zeros_c, ones128,
                         n, g, rpw, w_partial, psize)
    return sums, cnts


def kernel(aff_x, aff_adj, drug_x, drug_edge_index, drug_batch,
           target_x, target_edge_index, target_batch, params, num_drug, num_target):
    nd_g, nt_g = 1000, 1500
    n_d, n_t = drug_x.shape[0], target_x.shape[0]
    p = params

    ones128 = jnp.ones((128, 16), F32)
    ones512 = jnp.ones((_IW, 16), F32)
    zeros512_16 = jnp.zeros((512, 16), F32)
    zeros512_32b = jnp.zeros((512, 32), BF16)
    zeros512_64b = jnp.zeros((512, 64), BF16)
    zeros_g = jnp.zeros((96, 256), F32)
    zeros_c = jnp.zeros((96, 16), F32)

    aff_emb = _dense_gcn_tc(aff_x, aff_adj, p)

    row2d, col2d = _pad_edges(drug_edge_index, n_d)
    row2t, col2t = _pad_edges(target_edge_index, n_t)
    cnt_d, cnt_t = _deg_sc(col2d, col2t, ones512, zeros512_16, n_d, n_t)

    sums_d, cnts_d = _graph_path(
        drug_x, drug_edge_index, drug_batch, cnt_d,
        p['d_W0'], p['d_b0'], p['d_W1'], p['d_b1'], nd_g, n_d, 64, 4,
        1664, 30, 80, zeros512_64b, zeros_g, zeros_c, ones128)
    sums_t, cnts_t = _graph_path(
        target_x, target_edge_index, target_batch, cnt_t,
        p['t_W0'], p['t_b0'], p['t_W1'], p['t_b1'], nt_g, n_t, 32, 8,
        3200, 31, 32, zeros512_32b, zeros_g, zeros_c, ones128)

    drug_out = _contrast_tc(aff_emb[:nd_g], sums_d, cnts_d, p, 'dc_')
    target_out = _contrast_tc(aff_emb[nd_g:], sums_t, cnts_t, p, 'tc_')
    drug_out = drug_out + jnp.asarray(num_drug - nd_g, dtype=drug_out.dtype)
    target_out = target_out + jnp.asarray(num_target - nt_g, dtype=target_out.dtype)
    return (drug_out, target_out)


# speedup vs baseline: 5.3639x; 1.0855x over previous
import functools

import jax
import jax.numpy as jnp
from jax import lax
from jax.experimental import pallas as pl
from jax.experimental.pallas import tpu as pltpu
from jax.experimental.pallas import tpu_sc as plsc

F32 = jnp.float32
BF16 = jnp.bfloat16
I32 = jnp.int32
_NC = 2
_NS = 16
_IW = 512


def _sc_mesh():
    return plsc.VectorSubcoreMesh(core_axis_name="c", subcore_axis_name="s",
                                  num_cores=_NC, num_subcores=_NS)


def _rpt8(n):
    return -(-(-(-(n + 16) // 16)) // 8) * 8


def _deg_sc(col3_d, col3_t, ones128, zeros512, nd, nt):
    rpt_d = _rpt8(nd)
    rpt_t = _rpt8(nt)
    n_alloc = 16 * max(rpt_d, rpt_t)
    gd = col3_d.shape[0] // _NS
    gt = col3_t.shape[0] // _NS

    @functools.partial(
        pl.kernel, mesh=_sc_mesh(),
        compiler_params=pltpu.CompilerParams(use_tc_tiling_on_sc=False),
        out_type=[jax.ShapeDtypeStruct((nd, 16), F32),
                  jax.ShapeDtypeStruct((nt, 16), F32)],
        scratch_types=[
            pltpu.VMEM((_IW,), I32),
            pltpu.VMEM((_IW, 16), F32),
            pltpu.VMEM_SHARED((n_alloc, 16), F32),
        ],
    )
    def k(cd_hbm, ct_hbm, ones_hbm, z_hbm, dd_hbm, dt_hbm, idx_v, ones_v, acc):
        c = lax.axis_index("c")
        s = lax.axis_index("s")
        pltpu.sync_copy(ones_hbm, ones_v)
        for n, rpt, col_hbm, ngrp, out_hbm, core in (
                (nd, rpt_d, cd_hbm, gd, dd_hbm, 0),
                (nt, rpt_t, ct_hbm, gt, dt_hbm, 1)):
            @pl.when(c == core)
            def _():
                row0 = s * rpt
                zfull, ztail = rpt // 512, rpt % 512
                for zi in range(zfull):
                    pltpu.sync_copy(z_hbm,
                                    acc.at[pl.ds(row0 + zi * 512, 512)])
                if ztail:
                    pltpu.sync_copy(z_hbm.at[pl.ds(0, ztail)],
                                    acc.at[pl.ds(row0 + zfull * 512, ztail)])
                plsc.subcore_barrier()

                def grp(g, carry):
                    pltpu.sync_copy(col_hbm.at[s * ngrp + g], idx_v)
                    pltpu.sync_copy(ones_v, acc.at[idx_v], add=True)
                    return carry
                lax.fori_loop(0, ngrp, grp, 0)
                plsc.subcore_barrier()

                last = n - 15 * rpt
                @pl.when(s < _NS - 1)
                def _():
                    pltpu.sync_copy(acc.at[pl.ds(row0, rpt)],
                                    out_hbm.at[pl.ds(row0, rpt)])
                @pl.when(s == _NS - 1)
                def _():
                    pltpu.sync_copy(acc.at[pl.ds(row0, last)],
                                    out_hbm.at[pl.ds(row0, last)])
                plsc.subcore_barrier()

    return k(col3_d, col3_t, ones128, zeros512)


def _spmm_sc(p2d, row3, col3, zeros512, n, f, nchunk):
    rpt = _rpt8(n)
    n_alloc = 16 * rpt
    ngrp = row3.shape[0] // _NS
    npass = nchunk // _NC

    @functools.partial(
        pl.kernel, mesh=_sc_mesh(),
        compiler_params=pltpu.CompilerParams(use_tc_tiling_on_sc=False),
        out_type=jax.ShapeDtypeStruct((nchunk * n, f), BF16),
        scratch_types=[
            pltpu.VMEM((_IW,), I32),
            pltpu.VMEM((_IW,), I32),
            pltpu.VMEM((_IW,), I32),
            pltpu.VMEM((_IW,), I32),
            pltpu.VMEM((_IW, f), BF16),
            pltpu.VMEM((_IW, f), BF16),
            pltpu.VMEM_SHARED((n_alloc, f), BF16),
            pltpu.SemaphoreType.DMA,
            pltpu.SemaphoreType.DMA,
        ],
    )
    def k(p_hbm, row_hbm, col_hbm, z_hbm, s_hbm,
          idxr0, idxc0, idxr1, idxc1, buf0, buf1, acc, sem0, sem1):
        c = lax.axis_index("c")
        s = lax.axis_index("s")
        row0 = s * rpt
        zfull, ztail = rpt // 512, rpt % 512
        last = n - 15 * rpt

        def one_pass(ppass, carry):
            chunk = ppass * _NC + c
            off = chunk * n
            offv = jnp.full((16,), off, I32)
            for zi in range(zfull):
                pltpu.sync_copy(z_hbm, acc.at[pl.ds(row0 + zi * 512, 512)])
            if ztail:
                pltpu.sync_copy(z_hbm.at[pl.ds(0, ztail)],
                                acc.at[pl.ds(row0 + zfull * 512, ztail)])
            plsc.subcore_barrier()

            def fire(g, idxr, idxc, buf, sem):
                base = s * ngrp + g
                pltpu.sync_copy(row_hbm.at[base], idxr)
                pltpu.sync_copy(col_hbm.at[base], idxc)
                for jj in range(_IW // 16):
                    sl = pl.ds(jj * 16, 16)
                    idxr[sl] = idxr[sl] + offv
                pltpu.async_copy(p_hbm.at[idxr], buf, sem)

            def drain_scatter(idxr, idxc, buf, sem):
                pltpu.make_async_copy(p_hbm.at[idxr], buf, sem).wait()
                pltpu.sync_copy(buf, acc.at[idxc], add=True)

            fire(0, idxr0, idxc0, buf0, sem0)

            def grp2(g2, carry2):
                fire(2 * g2 + 1, idxr1, idxc1, buf1, sem1)
                drain_scatter(idxr0, idxc0, buf0, sem0)

                @pl.when(2 * g2 + 2 < ngrp)
                def _():
                    fire(2 * g2 + 2, idxr0, idxc0, buf0, sem0)
                drain_scatter(idxr1, idxc1, buf1, sem1)
                return carry2
            lax.fori_loop(0, ngrp // 2, grp2, 0)
            if ngrp % 2:
                drain_scatter(idxr0, idxc0, buf0, sem0)
            plsc.subcore_barrier()

            @pl.when(s < _NS - 1)
            def _():
                pltpu.sync_copy(acc.at[pl.ds(row0, rpt)],
                                s_hbm.at[pl.ds(off + row0, rpt)])
            @pl.when(s == _NS - 1)
            def _():
                pltpu.sync_copy(acc.at[pl.ds(row0, last)],
                                s_hbm.at[pl.ds(off + row0, last)])
            plsc.subcore_barrier()
            return carry

        lax.fori_loop(0, npass, one_pass, 0)

    return k(p2d, row3, col3, zeros512)


def _gmp_sc(h, batch3, zeros_g, zeros_c, ones128, n, g, rpw, w_partial, psize):
    g_pt = _rpt8(g)
    g_alloc = 16 * g_pt
    g_up = -(-g // 8) * 8
    nf_max = rpw // 128

    @functools.partial(
        pl.kernel, mesh=_sc_mesh(),
        compiler_params=pltpu.CompilerParams(use_tc_tiling_on_sc=False),
        out_type=[jax.ShapeDtypeStruct((_NC * g_up, 256), F32),
                  jax.ShapeDtypeStruct((_NC * g_up, 16), F32)],
        scratch_types=[
            pltpu.VMEM((128, 256), F32),
            pltpu.VMEM((128, 16), F32),
            pltpu.VMEM((nf_max, 128), I32),
            pltpu.VMEM_SHARED((g_alloc, 256), F32),
            pltpu.VMEM_SHARED((g_alloc, 16), F32),
        ],
    )
    def k(h_hbm, b_hbm, zg_hbm, zc_hbm, ones_hbm, s_hbm, c_hbm,
          buf, ones_v, idxb, accs, accc):
        c = lax.axis_index("c")
        s = lax.axis_index("s")
        w = c * _NS + s
        pltpu.sync_copy(ones_hbm, ones_v)
        grow0 = s * g_pt
        pltpu.sync_copy(zg_hbm.at[pl.ds(0, g_pt)], accs.at[pl.ds(grow0, g_pt)])
        pltpu.sync_copy(zc_hbm.at[pl.ds(0, g_pt)], accc.at[pl.ds(grow0, g_pt)])
        plsc.subcore_barrier()

        base = w * rpw
        nfull = jnp.clip((n - base) // 128, 0, nf_max)
        pltpu.sync_copy(b_hbm.at[w], idxb)

        def chunk(j, carry):
            r0 = base + j * 128
            pltpu.sync_copy(h_hbm.at[pl.ds(r0, 128)], buf)
            pltpu.sync_copy(buf, accs.at[idxb.at[j]], add=True)
            pltpu.sync_copy(ones_v, accc.at[idxb.at[j]], add=True)
            return carry
        lax.fori_loop(0, nfull, chunk, 0)

        if psize:
            p_j = (n - w_partial * rpw) // 128

            @pl.when(w == w_partial)
            def _():
                r0 = w_partial * rpw + p_j * 128
                pltpu.sync_copy(h_hbm.at[pl.ds(r0, psize)],
                                buf.at[pl.ds(0, psize)])
                pltpu.sync_copy(buf, accs.at[idxb.at[p_j]], add=True)
                pltpu.sync_copy(ones_v, accc.at[idxb.at[p_j]], add=True)
        plsc.subcore_barrier()

        glast = g - 15 * g_pt
        @pl.when(s < _NS - 1)
        def _():
            pltpu.sync_copy(accs.at[pl.ds(grow0, g_pt)],
                            s_hbm.at[pl.ds(c * g_up + grow0, g_pt)])
            pltpu.sync_copy(accc.at[pl.ds(grow0, g_pt)],
                            c_hbm.at[pl.ds(c * g_up + grow0, g_pt)])
        @pl.when(s == _NS - 1)
        def _():
            pltpu.sync_copy(accs.at[pl.ds(grow0, glast)],
                            s_hbm.at[pl.ds(c * g_up + grow0, glast)])
            pltpu.sync_copy(accc.at[pl.ds(grow0, glast)],
                            c_hbm.at[pl.ds(c * g_up + grow0, glast)])

    return k(h, batch3, zeros_g, zeros_c, ones128)


def _rowsum_body(a_ref, o_ref):
    o_ref[...] = jnp.broadcast_to(jnp.sum(a_ref[...], axis=1, keepdims=True),
                                  o_ref.shape)


def _aff_rowsum(a_pad):
    m = a_pad.shape[0]
    bm = 256
    return pl.pallas_call(
        _rowsum_body,
        grid=(m // bm,),
        in_specs=[pl.BlockSpec((bm, m), lambda i: (i, 0))],
        out_specs=pl.BlockSpec((bm, 8), lambda i: (i, 0)),
        out_shape=jax.ShapeDtypeStruct((m, 8), F32),
    )(a_pad)


def _mm_scale_body(x_ref, w_ref, s_ref, o_ref):
    d = lax.rsqrt(jnp.clip(s_ref[...][:, :1], 1.0, None))
    o_ref[...] = jnp.dot(x_ref[...], w_ref[...], preferred_element_type=F32) * d


def _mm_scale(x, w, s, bm=256):
    m, kk = x.shape
    n = w.shape[1]
    return pl.pallas_call(
        _mm_scale_body,
        grid=(m // bm,),
        in_specs=[
            pl.BlockSpec((bm, kk), lambda i: (i, 0)),
            pl.BlockSpec((kk, n), lambda i: (0, 0)),
            pl.BlockSpec((bm, 8), lambda i: (i, 0)),
        ],
        out_specs=pl.BlockSpec((bm, n), lambda i: (i, 0)),
        out_shape=jax.ShapeDtypeStruct((m, n), F32),
    )(x, w, s)


def _amm_body(a_ref, y_ref, s_ref, b_ref, o_ref):
    t = jnp.dot(a_ref[...], y_ref[...], preferred_element_type=F32)
    d = lax.rsqrt(jnp.clip(s_ref[...][:, :1], 1.0, None))
    o_ref[...] = jax.nn.relu(t * d + b_ref[...])


def _amm(a, y, s, b, bm=256):
    m = a.shape[0]
    kk, n = y.shape
    return pl.pallas_call(
        _amm_body,
        grid=(m // bm,),
        in_specs=[
            pl.BlockSpec((bm, kk), lambda i: (i, 0)),
            pl.BlockSpec((kk, n), lambda i: (0, 0)),
            pl.BlockSpec((bm, 8), lambda i: (i, 0)),
            pl.BlockSpec((1, n), lambda i: (0, 0)),
        ],
        out_specs=pl.BlockSpec((bm, n), lambda i: (i, 0)),
        out_shape=jax.ShapeDtypeStruct((m, n), F32),
    )(a, y, s, b)


def _dense_gcn_tc(aff_x, aff_adj, p):
    n = aff_adj.shape[0]
    m = ((n + 255) // 256) * 256
    a_pad = jnp.pad(aff_adj, ((0, m - n), (0, m - n)))
    x_pad = jnp.pad(aff_x, ((0, m - n), (0, 0)))
    s = _aff_rowsum(a_pad)
    y = _mm_scale(x_pad, p['aff_W0'], s)
    h1 = _amm(a_pad, y, s, p['aff_b0'].reshape(1, -1))
    y2 = _mm_scale(h1, p['aff_W1'], s)
    h2 = _amm(a_pad, y2, s, p['aff_b1'].reshape(1, -1))
    return h2[:n]


def _node_mm_body(x_ref, w_ref, cnt_ref, o_ref, *, nchunk, f):
    d = lax.rsqrt(cnt_ref[...][:, :1] + 1.0)
    t = jnp.dot(x_ref[...], w_ref[...], preferred_element_type=F32) * d
    for kk in range(nchunk):
        o_ref[kk] = t[:, kk * f:(kk + 1) * f].astype(BF16)


def _node_mm(x, w, cnt, nchunk, f, bm=2000):
    n, kdim = x.shape
    body = functools.partial(_node_mm_body, nchunk=nchunk, f=f)
    out = pl.pallas_call(
        body,
        grid=(n // bm,),
        in_specs=[
            pl.BlockSpec((bm, kdim), lambda i: (i, 0)),
            pl.BlockSpec((kdim, w.shape[1]), lambda i: (0, 0)),
            pl.BlockSpec((bm, 16), lambda i: (i, 0)),
        ],
        out_specs=pl.BlockSpec((nchunk, bm, f), lambda i: (0, i, 0)),
        out_shape=jax.ShapeDtypeStruct((nchunk, n, f), BF16),
    )(x, w, cnt)
    return out.reshape(nchunk * n, f)


def _layer2_body(s3_ref, p3_ref, cnt_ref, w_ref, b_ref, o_ref, *, nchunk, f):
    d = lax.rsqrt(cnt_ref[...][:, :1] + 1.0)
    h = jnp.concatenate(
        [s3_ref[kk].astype(F32) + p3_ref[kk].astype(F32) for kk in range(nchunk)],
        axis=1)
    h = jax.nn.relu(h * d + b_ref[...])
    t = jnp.dot(h, w_ref[...], preferred_element_type=F32) * d
    for kk in range(nchunk):
        o_ref[kk] = t[:, kk * f:(kk + 1) * f].astype(BF16)


def _layer2_mm(s2d, p2d, cnt, w, b, nchunk, f, bm=2000):
    n = cnt.shape[0]
    s3 = s2d.reshape(nchunk, n, f)
    p3 = p2d.reshape(nchunk, n, f)
    body = functools.partial(_layer2_body, nchunk=nchunk, f=f)
    out = pl.pallas_call(
        body,
        grid=(n // bm,),
        in_specs=[
            pl.BlockSpec((nchunk, bm, f), lambda i: (0, i, 0)),
            pl.BlockSpec((nchunk, bm, f), lambda i: (0, i, 0)),
            pl.BlockSpec((bm, 16), lambda i: (i, 0)),
            pl.BlockSpec((w.shape[0], w.shape[1]), lambda i: (0, 0)),
            pl.BlockSpec((1, w.shape[1]), lambda i: (0, 0)),
        ],
        out_specs=pl.BlockSpec((nchunk, bm, f), lambda i: (0, i, 0)),
        out_shape=jax.ShapeDtypeStruct((nchunk, n, f), BF16),
    )(s3, p3, cnt, w, b.reshape(1, -1))
    return out.reshape(nchunk * n, f)


def _relu_comb_body(s3_ref, p3_ref, cnt_ref, b_ref, o_ref, *, nchunk, f):
    d = lax.rsqrt(cnt_ref[...][:, :1] + 1.0)
    h = jnp.concatenate(
        [s3_ref[kk].astype(F32) + p3_ref[kk].astype(F32) for kk in range(nchunk)],
        axis=1)
    o_ref[...] = jax.nn.relu(h * d + b_ref[...])


def _relu_comb(s2d, p2d, cnt, b, nchunk, f, bm=2000):
    n = cnt.shape[0]
    s3 = s2d.reshape(nchunk, n, f)
    p3 = p2d.reshape(nchunk, n, f)
    body = functools.partial(_relu_comb_body, nchunk=nchunk, f=f)
    return pl.pallas_call(
        body,
        grid=(n // bm,),
        in_specs=[
            pl.BlockSpec((nchunk, bm, f), lambda i: (0, i, 0)),
            pl.BlockSpec((nchunk, bm, f), lambda i: (0, i, 0)),
            pl.BlockSpec((bm, 16), lambda i: (i, 0)),
            pl.BlockSpec((1, nchunk * f), lambda i: (0, 0)),
        ],
        out_specs=pl.BlockSpec((bm, nchunk * f), lambda i: (i, 0)),
        out_shape=jax.ShapeDtypeStruct((n, nchunk * f), F32),
    )(s3, p3, cnt, b.reshape(1, -1))


def _elu(x):
    return jnp.where(x > 0, x, jnp.exp(x) - 1.0)


def _contrast_body(za_ref, sums_ref, cnts_ref, p1_ref, pb1_ref, p2_ref, pb2_ref,
                   wq_ref, bq_ref, wk_ref, bk_ref, wv_ref, bv_ref, o_ref):
    g = za_ref.shape[0]
    g_up = sums_ref.shape[0] // 2
    sums = sums_ref[...]
    cnts = cnts_ref[...]
    cnt = jnp.clip(cnts[:g, :1] + cnts[g_up:g_up + g, :1], 1.0, None)
    zb = (sums[:g] + sums[g_up:g_up + g]) / cnt
    za = za_ref[...]
    p1 = p1_ref[...]
    p2 = p2_ref[...]
    za_p = jnp.dot(_elu(jnp.dot(za, p1, preferred_element_type=F32) + pb1_ref[...]),
                   p2, preferred_element_type=F32) + pb2_ref[...]
    zb_p = jnp.dot(_elu(jnp.dot(zb, p1, preferred_element_type=F32) + pb1_ref[...]),
                   p2, preferred_element_type=F32) + pb2_ref[...]
    scale = jnp.sqrt(za_p.shape[1] / 2.0).astype(F32)

    def ca(q_in, k_in, v_in):
        q = jnp.dot(q_in, wq_ref[...], preferred_element_type=F32) + bq_ref[...]
        k = jnp.dot(k_in, wk_ref[...], preferred_element_type=F32) + bk_ref[...]
        v = jnp.dot(v_in, wv_ref[...], preferred_element_type=F32) + bv_ref[...]
        logits = lax.dot_general(q, k, (((1,), (1,)), ((), ())),
                                 preferred_element_type=F32) / scale
        logits = logits - jnp.max(logits, axis=-1, keepdims=True)
        e = jnp.exp(logits)
        a = e / jnp.sum(e, axis=-1, keepdims=True)
        return jnp.dot(a, v, preferred_element_type=F32)

    o_ref[...] = jnp.concatenate([ca(za_p, zb_p, zb_p), ca(zb_p, za_p, za_p)], axis=1)


def _contrast_tc(za, sums, cnts, p, pre):
    m = za.shape[0]
    args = (za, sums, cnts, p[pre + 'P1'], p[pre + 'pb1'].reshape(1, -1),
            p[pre + 'P2'], p[pre + 'pb2'].reshape(1, -1),
            p[pre + 'Wq'], p[pre + 'bq'].reshape(1, -1),
            p[pre + 'Wk'], p[pre + 'bk'].reshape(1, -1),
            p[pre + 'Wv'], p[pre + 'bv'].reshape(1, -1))
    return pl.pallas_call(
        _contrast_body,
        out_shape=jax.ShapeDtypeStruct((m, 256), F32),
    )(*args)


def _pad_edges(edge_index, n):
    e = edge_index.shape[1]
    ept = -(-e // (_NS * _IW)) * _IW
    e_pad = _NS * ept
    pad = e_pad - e
    row = jnp.concatenate([edge_index[0],
                           jnp.arange(pad, dtype=I32) % jnp.int32(n)])
    col = jnp.concatenate([edge_index[1], jnp.full((pad,), n, I32)])
    return row.reshape(-1, _IW), col.reshape(-1, _IW)


def _pad_batch(batch, n, g, rpw):
    npad = 32 * rpw - n
    return jnp.concatenate([batch, jnp.full((npad,), g, I32)]
                           ).reshape(32, rpw // 128, 128)


def _graph_path(x, edge_index, batch, cnt, w0, b0, w1, b1, g, n, f, nchunk,
                rpw, w_partial, psize, zeros512f, zeros_g, zeros_c, ones128):
    row2, col2 = _pad_edges(edge_index, n)
    p1 = _node_mm(x, w0, cnt, nchunk, f)
    s1 = _spmm_sc(p1, row2, col2, zeros512f, n, f, nchunk)
    p2 = _layer2_mm(s1, p1, cnt, w1, b0, nchunk, f)
    s2 = _spmm_sc(p2, row2, col2, zeros512f, n, f, nchunk)
    h2 = _relu_comb(s2, p2, cnt, b1, nchunk, f)
    batch2 = _pad_batch(batch, n, g, rpw)
    sums, cnts = _gmp_sc(h2, batch2, zeros_g, zeros_c, ones128,
                         n, g, rpw, w_partial, psize)
    return sums, cnts


def kernel(aff_x, aff_adj, drug_x, drug_edge_index, drug_batch,
           target_x, target_edge_index, target_batch, params, num_drug, num_target):
    nd_g, nt_g = 1000, 1500
    n_d, n_t = drug_x.shape[0], target_x.shape[0]
    p = params

    ones128 = jnp.ones((128, 16), F32)
    ones512 = jnp.ones((_IW, 16), F32)
    zeros512_16 = jnp.zeros((512, 16), F32)
    zeros512_32b = jnp.zeros((512, 32), BF16)
    zeros_g = jnp.zeros((96, 256), F32)
    zeros_c = jnp.zeros((96, 16), F32)

    aff_emb = _dense_gcn_tc(aff_x, aff_adj, p)

    row2d, col2d = _pad_edges(drug_edge_index, n_d)
    row2t, col2t = _pad_edges(target_edge_index, n_t)
    cnt_d, cnt_t = _deg_sc(col2d, col2t, ones512, zeros512_16, n_d, n_t)

    sums_d, cnts_d = _graph_path(
        drug_x, drug_edge_index, drug_batch, cnt_d,
        p['d_W0'], p['d_b0'], p['d_W1'], p['d_b1'], nd_g, n_d, 32, 8,
        1664, 30, 80, zeros512_32b, zeros_g, zeros_c, ones128)
    sums_t, cnts_t = _graph_path(
        target_x, target_edge_index, target_batch, cnt_t,
        p['t_W0'], p['t_b0'], p['t_W1'], p['t_b1'], nt_g, n_t, 32, 8,
        3200, 31, 32, zeros512_32b, zeros_g, zeros_c, ones128)

    drug_out = _contrast_tc(aff_emb[:nd_g], sums_d, cnts_d, p, 'dc_')
    target_out = _contrast_tc(aff_emb[nd_g:], sums_t, cnts_t, p, 'tc_')
    drug_out = drug_out + jnp.asarray(num_drug - nd_g, dtype=drug_out.dtype)
    target_out = target_out + jnp.asarray(num_target - nt_g, dtype=target_out.dtype)
    return (drug_out, target_out)
